# Initial kernel scaffold; baseline (speedup 1.0000x reference)
import jax
import jax.numpy as jnp
from jax.experimental import pallas as pl


def kernel(x, edge_index, edge_attr, W1, b1, W2, b2, W3, b3, M1w, M1b, M2w, M2b, M3w, M3b):
    raise NotImplementedError("write your pallas kernel here")


# speedup vs baseline: 2.9020x; 2.9020x over previous
import functools

import jax
import jax.numpy as jnp
from jax import lax
from jax.experimental import pallas as pl
from jax.experimental.pallas import tpu as pltpu
from jax.experimental.pallas import tpu_sc as plsc

N = 50000
E = 800000
DIN = 200
H = 100
C = 11

NB = 4
BW = 32
CH = 128
NS = 16
NC = 2
ROWS_PT = N // NS
EPAD = 802816
DEG_CHUNKS = EPAD // (NC * NS * CH)
SC_CHUNKS = EPAD // (NS * CH)

RB = 2000
GRID = N // RB

f32 = jnp.float32
i32 = jnp.int32


@functools.cache
def _mesh():
    return plsc.VectorSubcoreMesh(core_axis_name="c", subcore_axis_name="s")


def _copy_1d_slices(s, src, stage, dst):
    r0 = pl.multiple_of(s * 3128, 8)

    @pl.when(s < NS - 1)
    def _():
        pltpu.sync_copy(src.at[pl.ds(r0, 3128)], stage.at[pl.ds(0, 3128)])
        pltpu.sync_copy(stage.at[pl.ds(0, 3128)], dst.at[pl.ds(r0, 3128)])

    @pl.when(s == NS - 1)
    def _():
        r1 = 3128 * (NS - 1)
        nr = N - r1
        pltpu.sync_copy(src.at[pl.ds(r1, nr)], stage.at[pl.ds(0, nr)])
        pltpu.sync_copy(stage.at[pl.ds(0, nr)], dst.at[pl.ds(r1, nr)])


def _sc_deg_body(dst_hbm, ea_hbm, z1_hbm, dega, degb, acc, dst_v, ea_v, stage):
    c = lax.axis_index("c")
    s = lax.axis_index("s")
    gtid = c * NS + s
    _copy_1d_slices(s, z1_hbm, stage, acc)
    plsc.subcore_barrier()

    def chunk(i, carry):
        base = pl.multiple_of((gtid * DEG_CHUNKS + i) * CH, CH)
        pltpu.sync_copy(dst_hbm.at[pl.ds(base, CH)], dst_v)
        pltpu.sync_copy(ea_hbm.at[pl.ds(base, CH)], ea_v)
        pltpu.sync_copy(ea_v, acc.at[dst_v], add=True)
        return carry

    lax.fori_loop(0, DEG_CHUNKS, chunk, 0)
    plsc.subcore_barrier()

    @pl.when(c == 0)
    def _():
        _copy_1d_slices(s, acc, stage, dega)

    @pl.when(c == 1)
    def _():
        _copy_1d_slices(s, acc, stage, degb)


STG = 392


def _staged_rows(src, stage, dst, r0, nrows):
    for off in range(0, nrows, STG):
        n = min(STG, nrows - off)
        ro = pl.multiple_of(r0 + off, 8)
        pltpu.sync_copy(src.at[pl.ds(ro, n)], stage.at[pl.ds(0, n)])
        pltpu.sync_copy(stage.at[pl.ds(0, n)], dst.at[pl.ds(ro, n)])


def _copy_2d_slices(s, src, stage, dst):
    @pl.when(s < NS - 1)
    def _():
        _staged_rows(src, stage, dst, s * 3128, 3128)

    @pl.when(s == NS - 1)
    def _():
        _staged_rows(src, stage, dst, 3128 * (NS - 1), N - 3128 * (NS - 1))


def _sc_scatter_body(src_hbm, dst_hbm, ea_hbm, z_hbm, gj, sj, s_idx,
                     acc, src_v, dst_v, ea_v, rows_v, stage, sem):
    _copy_2d_slices(s_idx, z_hbm, stage, acc)
    plsc.subcore_barrier()

    iota_lo = lax.iota(i32, 16)
    iota_hi = iota_lo + 16

    def chunk(i, carry):
        base = pl.multiple_of((s_idx * SC_CHUNKS + i) * CH, CH)
        pltpu.sync_copy(src_hbm.at[pl.ds(base, CH)], src_v)
        pltpu.sync_copy(dst_hbm.at[pl.ds(base, CH)], dst_v)
        pltpu.sync_copy(ea_hbm.at[pl.ds(base, CH)], ea_v)
        pltpu.async_copy(gj.at[src_v], rows_v, sem).wait()
        for e in range(CH):
            e_splat = jnp.full((16,), e, i32)
            sp = plsc.load_gather(ea_v, [e_splat])
            lo = plsc.load_gather(rows_v, [e_splat, iota_lo])
            plsc.store_scatter(rows_v, [e_splat, iota_lo], lo * sp)
            hi = plsc.load_gather(rows_v, [e_splat, iota_hi])
            plsc.store_scatter(rows_v, [e_splat, iota_hi], hi * sp)
        pltpu.sync_copy(rows_v, acc.at[dst_v], add=True)
        return carry

    lax.fori_loop(0, SC_CHUNKS, chunk, 0)
    plsc.subcore_barrier()
    _copy_2d_slices(s_idx, acc, stage, sj)
    plsc.subcore_barrier()


def _sc_scatter_kernel(g0, g1, g2, g3, src_hbm, dst_hbm, ea_hbm, z_hbm,
                       s0, s1, s2, s3, acc, src_v, dst_v, ea_v, rows_v,
                       stage, sem):
    c = lax.axis_index("c")
    s = lax.axis_index("s")

    @pl.when(c == 0)
    def _():
        _sc_scatter_body(src_hbm, dst_hbm, ea_hbm, z_hbm, g0, s0, s,
                         acc, src_v, dst_v, ea_v, rows_v, stage, sem)
        _sc_scatter_body(src_hbm, dst_hbm, ea_hbm, z_hbm, g1, s1, s,
                         acc, src_v, dst_v, ea_v, rows_v, stage, sem)

    @pl.when(c == 1)
    def _():
        _sc_scatter_body(src_hbm, dst_hbm, ea_hbm, z_hbm, g2, s2, s,
                         acc, src_v, dst_v, ea_v, rows_v, stage, sem)
        _sc_scatter_body(src_hbm, dst_hbm, ea_hbm, z_hbm, g3, s3, s,
                         acc, src_v, dst_v, ea_v, rows_v, stage, sem)


@functools.cache
def _sc_deg():
    return pl.kernel(
        _sc_deg_body,
        out_type=[jax.ShapeDtypeStruct((N,), f32),
                  jax.ShapeDtypeStruct((N,), f32)],
        mesh=_mesh(),
        scratch_types=[
            pltpu.VMEM_SHARED((N,), f32),
            pltpu.VMEM((CH,), i32),
            pltpu.VMEM((CH,), f32),
            pltpu.VMEM((3128,), f32),
        ],
        compiler_params=pltpu.CompilerParams(needs_layout_passes=False,
                                             use_tc_tiling_on_sc=False),
    )


@functools.cache
def _sc_scatter():
    return pl.kernel(
        _sc_scatter_kernel,
        out_type=[jax.ShapeDtypeStruct((N, BW), f32) for _ in range(NB)],
        mesh=_mesh(),
        scratch_types=[
            pltpu.VMEM_SHARED((N, BW), f32),
            pltpu.VMEM((CH,), i32),
            pltpu.VMEM((CH,), i32),
            pltpu.VMEM((CH,), f32),
            pltpu.VMEM((CH, BW), f32),
            pltpu.VMEM((STG, BW), f32),
            pltpu.SemaphoreType.DMA,
        ],
        compiler_params=pltpu.CompilerParams(needs_layout_passes=False,
                                             use_tc_tiling_on_sc=False),
    )


def _tc_first_body(dega_ref, degb_ref, x_ref, w_ref, dinv_ref,
                   g0_ref, g1_ref, g2_ref, g3_ref):
    deg = dega_ref[...] + degb_ref[...] + 1.0
    dinv = lax.rsqrt(deg)
    dinv_ref[...] = dinv
    hw = jnp.dot(x_ref[...], w_ref[...], preferred_element_type=f32)
    gg = hw * dinv
    g0_ref[...] = gg[:, 0 * BW:1 * BW]
    g1_ref[...] = gg[:, 1 * BW:2 * BW]
    g2_ref[...] = gg[:, 2 * BW:3 * BW]
    g3_ref[...] = gg[:, 3 * BW:4 * BW]


def _tc_mid_body(s0, s1, s2, s3, g0, g1, g2, g3, dinv_ref, b_ref, w_ref,
                 o0, o1, o2, o3):
    dinv = dinv_ref[...]
    hcat = jnp.concatenate(
        [s0[...] + g0[...], s1[...] + g1[...], s2[...] + g2[...], s3[...] + g3[...]],
        axis=1)
    t = jnp.maximum(dinv * hcat + b_ref[...], 0.0)
    hw = jnp.dot(t, w_ref[...], preferred_element_type=f32)
    gg = hw * dinv
    o0[...] = gg[:, 0 * BW:1 * BW]
    o1[...] = gg[:, 1 * BW:2 * BW]
    o2[...] = gg[:, 2 * BW:3 * BW]
    o3[...] = gg[:, 3 * BW:4 * BW]


def _tc_head_body(s0, s1, s2, s3, g0, g1, g2, g3, dinv_ref, b_ref,
                  m1w, m1b, m2w, m2b, m3w, m3b, out_ref):
    dinv = dinv_ref[...]
    hcat = jnp.concatenate(
        [s0[...] + g0[...], s1[...] + g1[...], s2[...] + g2[...], s3[...] + g3[...]],
        axis=1)
    t = jnp.maximum(dinv * hcat + b_ref[...], 0.0)
    h3 = jnp.concatenate([t[:, j * BW:j * BW + 25] for j in range(NB)], axis=1)
    m1 = jnp.maximum(jnp.dot(h3, m1w[...], preferred_element_type=f32) + m1b[...], 0.0)
    m2 = jnp.maximum(jnp.dot(m1, m2w[...], preferred_element_type=f32) + m2b[...], 0.0)
    lg = jnp.dot(m2, m3w[...], preferred_element_type=f32) + m3b[...]
    mx = jnp.max(lg, axis=1, keepdims=True)
    lse = mx + jnp.log(jnp.sum(jnp.exp(lg - mx), axis=1, keepdims=True))
    out_ref[...] = lg - lse


def _row_spec(w):
    return pl.BlockSpec((RB, w), lambda i: (i, 0))


def _full_spec(shape):
    return pl.BlockSpec(shape, lambda i: tuple(0 for _ in shape))


def _tc_first(dega2, degb2, x, w1p):
    return pl.pallas_call(
        _tc_first_body,
        grid=(GRID,),
        in_specs=[_row_spec(1), _row_spec(1), _row_spec(DIN), _full_spec((DIN, NB * BW))],
        out_specs=[_row_spec(1)] + [_row_spec(BW)] * NB,
        out_shape=[jax.ShapeDtypeStruct((N, 1), f32)]
        + [jax.ShapeDtypeStruct((N, BW), f32) for _ in range(NB)],
    )(dega2, degb2, x, w1p)


def _tc_mid(ss, gs, dinv, bp, wp):
    return pl.pallas_call(
        _tc_mid_body,
        grid=(GRID,),
        in_specs=[_row_spec(BW)] * (2 * NB)
        + [_row_spec(1), _full_spec((1, NB * BW)), _full_spec((NB * BW, NB * BW))],
        out_specs=[_row_spec(BW)] * NB,
        out_shape=[jax.ShapeDtypeStruct((N, BW), f32) for _ in range(NB)],
    )(*ss, *gs, dinv, bp, wp)


def _tc_head(ss, gs, dinv, bp, m1w, m1b, m2w, m2b, m3w, m3b):
    return pl.pallas_call(
        _tc_head_body,
        grid=(GRID,),
        in_specs=[_row_spec(BW)] * (2 * NB)
        + [_row_spec(1), _full_spec((1, NB * BW)),
           _full_spec((H, H // 2)), _full_spec((1, H // 2)),
           _full_spec((H // 2, H // 2)), _full_spec((1, H // 2)),
           _full_spec((H // 2, C)), _full_spec((1, C))],
        out_specs=_row_spec(C),
        out_shape=jax.ShapeDtypeStruct((N, C), f32),
    )(*ss, *gs, dinv, bp, m1w, m1b, m2w, m2b, m3w, m3b)


def _pad_w_in(w):
    fi = w.shape[0]
    return jnp.pad(w.reshape(fi, NB, 25), ((0, 0), (0, 0), (0, BW - 25))).reshape(fi, NB * BW)


def _pad_w_both(w):
    w4 = w.reshape(NB, 25, NB, 25)
    w4 = jnp.pad(w4, ((0, 0), (0, BW - 25), (0, 0), (0, BW - 25)))
    return w4.reshape(NB * BW, NB * BW)


def _pad_b(b):
    return jnp.pad(b.reshape(NB, 25), ((0, 0), (0, BW - 25))).reshape(1, NB * BW)


def kernel(x, edge_index, edge_attr, W1, b1, W2, b2, W3, b3,
           M1w, M1b, M2w, M2b, M3w, M3b):
    src = edge_index[0].astype(i32)
    dst = edge_index[1].astype(i32)
    pad = EPAD - E
    srcp = jnp.concatenate([src, jnp.zeros((pad,), i32)])
    dstp = jnp.concatenate([dst, jnp.zeros((pad,), i32)])
    eap = jnp.concatenate([edge_attr.astype(f32), jnp.zeros((pad,), f32)])
    zeros1 = jnp.zeros((N,), f32)
    zeros32 = jnp.zeros((N, BW), f32)

    w1p = _pad_w_in(W1)
    w2p = _pad_w_both(W2)
    w3p = _pad_w_both(W3)
    b1p, b2p, b3p = _pad_b(b1), _pad_b(b2), _pad_b(b3)

    dega, degb = _sc_deg()(dstp, eap, zeros1)
    dinv, *g = _tc_first(dega.reshape(N, 1), degb.reshape(N, 1), x, w1p)

    s = _sc_scatter()(*g, srcp, dstp, eap, zeros32)
    g = _tc_mid(s, g, dinv, b1p, w2p)
    s = _sc_scatter()(*g, srcp, dstp, eap, zeros32)
    g = _tc_mid(s, g, dinv, b2p, w3p)
    s = _sc_scatter()(*g, srcp, dstp, eap, zeros32)

    return _tc_head(s, g, dinv, b3p,
                    M1w, M1b.reshape(1, -1), M2w, M2b.reshape(1, -1),
                    M3w, M3b.reshape(1, -1))


# speedup vs baseline: 6.6355x; 2.2865x over previous
import functools

import jax
import jax.numpy as jnp
from jax import lax
from jax.experimental import pallas as pl
from jax.experimental.pallas import tpu as pltpu
from jax.experimental.pallas import tpu_sc as plsc

N = 50000
E = 800000
DIN = 200
H = 100
C = 11

NB = 4
BW = 32
CH = 128
NS = 16
NC = 2
ROWS_PT = N // NS
EPAD = 802816
DEG_CHUNKS = EPAD // (NC * NS * CH)
SC_CHUNKS = EPAD // (NS * CH)

RB = 2000
GRID = N // RB

f32 = jnp.float32
i32 = jnp.int32


@functools.cache
def _mesh():
    return plsc.VectorSubcoreMesh(core_axis_name="c", subcore_axis_name="s")


def _copy_1d_slices(s, src, stage, dst):
    r0 = pl.multiple_of(s * 3128, 8)

    @pl.when(s < NS - 1)
    def _():
        pltpu.sync_copy(src.at[pl.ds(r0, 3128)], stage.at[pl.ds(0, 3128)])
        pltpu.sync_copy(stage.at[pl.ds(0, 3128)], dst.at[pl.ds(r0, 3128)])

    @pl.when(s == NS - 1)
    def _():
        r1 = 3128 * (NS - 1)
        nr = N - r1
        pltpu.sync_copy(src.at[pl.ds(r1, nr)], stage.at[pl.ds(0, nr)])
        pltpu.sync_copy(stage.at[pl.ds(0, nr)], dst.at[pl.ds(r1, nr)])


def _sc_deg_body(dst_hbm, ea_hbm, z1_hbm, dega, degb, acc, dst_v, ea_v, stage):
    c = lax.axis_index("c")
    s = lax.axis_index("s")
    gtid = c * NS + s
    _copy_1d_slices(s, z1_hbm, stage, acc)
    plsc.subcore_barrier()

    def chunk(i, carry):
        base = pl.multiple_of((gtid * DEG_CHUNKS + i) * CH, CH)
        pltpu.sync_copy(dst_hbm.at[pl.ds(base, CH)], dst_v)
        pltpu.sync_copy(ea_hbm.at[pl.ds(base, CH)], ea_v)
        pltpu.sync_copy(ea_v, acc.at[dst_v], add=True)
        return carry

    lax.fori_loop(0, DEG_CHUNKS, chunk, 0)
    plsc.subcore_barrier()

    @pl.when(c == 0)
    def _():
        _copy_1d_slices(s, acc, stage, dega)

    @pl.when(c == 1)
    def _():
        _copy_1d_slices(s, acc, stage, degb)


STG = 392


def _staged_rows(src, stage, dst, r0, nrows, dst_base=None):
    d0 = r0 if dst_base is None else dst_base
    for off in range(0, nrows, STG):
        n = min(STG, nrows - off)
        ro = pl.multiple_of(r0 + off, 8)
        do = pl.multiple_of(d0 + off, 8)
        pltpu.sync_copy(src.at[pl.ds(ro, n)], stage.at[pl.ds(0, n)])
        pltpu.sync_copy(stage.at[pl.ds(0, n)], dst.at[pl.ds(do, n)])


def _copy_2d_slices(s, src, stage, dst):
    @pl.when(s < NS - 1)
    def _():
        _staged_rows(src, stage, dst, s * 3128, 3128)

    @pl.when(s == NS - 1)
    def _():
        _staged_rows(src, stage, dst, 3128 * (NS - 1), N - 3128 * (NS - 1))


SUP = 8
SUPERS = SC_CHUNKS // SUP


def _sc_scatter_kernel(g_flat, src4_hbm, dst_hbm, ea_hbm, z_hbm, s_all,
                       acc, src_b, dst_b, ea_b, rows_v, stage, sem):
    c = lax.axis_index("c")
    s = lax.axis_index("s")

    def col_block(jj, carry0):
        j = c * 2 + jj
        _copy_2d_slices(s, z_hbm, stage, acc)
        plsc.subcore_barrier()

        def superchunk(u, carry1):
            row0 = s * SC_CHUNKS + u * SUP
            pltpu.sync_copy(src4_hbm.at[j, pl.ds(row0, SUP)], src_b)
            pltpu.sync_copy(dst_hbm.at[pl.ds(row0, SUP)], dst_b)
            pltpu.sync_copy(ea_hbm.at[pl.ds(row0, SUP)], ea_b)
            pltpu.async_copy(g_flat.at[src_b.at[0]], rows_v.at[0], sem)

            def chunk(k, carry2):
                b = lax.rem(k, 2)

                @pl.when(k < SUP - 1)
                def _():
                    pltpu.async_copy(g_flat.at[src_b.at[k + 1]],
                                     rows_v.at[lax.rem(k + 1, 2)], sem)

                pltpu.make_async_copy(g_flat.at[src_b.at[0]],
                                      rows_v.at[0], sem).wait()
                for e in range(CH):
                    e_splat = jnp.full((16,), e, i32)
                    k_splat = jnp.full((16,), k, i32)
                    sp = plsc.load_gather(ea_b, [k_splat, e_splat])
                    lo = rows_v[b, e, pl.ds(0, 16)]
                    hi = rows_v[b, e, pl.ds(16, 16)]
                    rows_v[b, e, pl.ds(0, 16)] = lo * sp
                    rows_v[b, e, pl.ds(16, 16)] = hi * sp
                pltpu.sync_copy(rows_v.at[b], acc.at[dst_b.at[k]], add=True)
                return carry2

            lax.fori_loop(0, SUP, chunk, 0)
            return carry1

        lax.fori_loop(0, SUPERS, superchunk, 0)
        plsc.subcore_barrier()

        @pl.when(s < NS - 1)
        def _():
            _staged_rows(acc, stage, s_all, s * 3128, 3128,
                         dst_base=j * N + s * 3128)

        @pl.when(s == NS - 1)
        def _():
            _staged_rows(acc, stage, s_all, 3128 * (NS - 1),
                         N - 3128 * (NS - 1),
                         dst_base=j * N + 3128 * (NS - 1))

        plsc.subcore_barrier()
        return carry0

    lax.fori_loop(0, 2, col_block, 0)


@functools.cache
def _sc_deg():
    return pl.kernel(
        _sc_deg_body,
        out_type=[jax.ShapeDtypeStruct((N,), f32),
                  jax.ShapeDtypeStruct((N,), f32)],
        mesh=_mesh(),
        scratch_types=[
            pltpu.VMEM_SHARED((N,), f32),
            pltpu.VMEM((CH,), i32),
            pltpu.VMEM((CH,), f32),
            pltpu.VMEM((3128,), f32),
        ],
        compiler_params=pltpu.CompilerParams(needs_layout_passes=False,
                                             use_tc_tiling_on_sc=False),
    )


@functools.cache
def _sc_scatter():
    return pl.kernel(
        _sc_scatter_kernel,
        out_type=jax.ShapeDtypeStruct((NB * N, BW), f32),
        mesh=_mesh(),
        scratch_types=[
            pltpu.VMEM_SHARED((N, BW), f32),
            pltpu.VMEM((SUP, CH), i32),
            pltpu.VMEM((SUP, CH), i32),
            pltpu.VMEM((SUP, CH), f32),
            pltpu.VMEM((2, CH, BW), f32),
            pltpu.VMEM((STG, BW), f32),
            pltpu.SemaphoreType.DMA,
        ],
        compiler_params=pltpu.CompilerParams(needs_layout_passes=False,
                                             use_tc_tiling_on_sc=False),
    )


def _tc_first_body(dega_ref, degb_ref, x_ref, w_ref, dinv_ref,
                   g0_ref, g1_ref, g2_ref, g3_ref):
    deg = dega_ref[...] + degb_ref[...] + 1.0
    dinv = lax.rsqrt(deg)
    dinv_ref[...] = dinv
    hw = jnp.dot(x_ref[...], w_ref[...], preferred_element_type=f32)
    gg = hw * dinv
    g0_ref[...] = gg[:, 0 * BW:1 * BW]
    g1_ref[...] = gg[:, 1 * BW:2 * BW]
    g2_ref[...] = gg[:, 2 * BW:3 * BW]
    g3_ref[...] = gg[:, 3 * BW:4 * BW]


def _tc_mid_body(s0, s1, s2, s3, g0, g1, g2, g3, dinv_ref, b_ref, w_ref,
                 o0, o1, o2, o3):
    dinv = dinv_ref[...]
    hcat = jnp.concatenate(
        [s0[...] + g0[...], s1[...] + g1[...], s2[...] + g2[...], s3[...] + g3[...]],
        axis=1)
    t = jnp.maximum(dinv * hcat + b_ref[...], 0.0)
    hw = jnp.dot(t, w_ref[...], preferred_element_type=f32)
    gg = hw * dinv
    o0[...] = gg[:, 0 * BW:1 * BW]
    o1[...] = gg[:, 1 * BW:2 * BW]
    o2[...] = gg[:, 2 * BW:3 * BW]
    o3[...] = gg[:, 3 * BW:4 * BW]


def _tc_head_body(s0, s1, s2, s3, g0, g1, g2, g3, dinv_ref, b_ref,
                  m1w, m1b, m2w, m2b, m3w, m3b, out_ref):
    dinv = dinv_ref[...]
    hcat = jnp.concatenate(
        [s0[...] + g0[...], s1[...] + g1[...], s2[...] + g2[...], s3[...] + g3[...]],
        axis=1)
    t = jnp.maximum(dinv * hcat + b_ref[...], 0.0)
    h3 = jnp.concatenate([t[:, j * BW:j * BW + 25] for j in range(NB)], axis=1)
    m1 = jnp.maximum(jnp.dot(h3, m1w[...], preferred_element_type=f32) + m1b[...], 0.0)
    m2 = jnp.maximum(jnp.dot(m1, m2w[...], preferred_element_type=f32) + m2b[...], 0.0)
    lg = jnp.dot(m2, m3w[...], preferred_element_type=f32) + m3b[...]
    mx = jnp.max(lg, axis=1, keepdims=True)
    lse = mx + jnp.log(jnp.sum(jnp.exp(lg - mx), axis=1, keepdims=True))
    out_ref[...] = lg - lse


def _row_spec(w):
    return pl.BlockSpec((RB, w), lambda i: (i, 0))


def _full_spec(shape):
    return pl.BlockSpec(shape, lambda i: tuple(0 for _ in shape))


def _tc_first(dega2, degb2, x, w1p):
    return pl.pallas_call(
        _tc_first_body,
        grid=(GRID,),
        in_specs=[_row_spec(1), _row_spec(1), _row_spec(DIN), _full_spec((DIN, NB * BW))],
        out_specs=[_row_spec(1)] + [_row_spec(BW)] * NB,
        out_shape=[jax.ShapeDtypeStruct((N, 1), f32)]
        + [jax.ShapeDtypeStruct((N, BW), f32) for _ in range(NB)],
    )(dega2, degb2, x, w1p)


def _tc_mid(ss, gs, dinv, bp, wp):
    return pl.pallas_call(
        _tc_mid_body,
        grid=(GRID,),
        in_specs=[_row_spec(BW)] * (2 * NB)
        + [_row_spec(1), _full_spec((1, NB * BW)), _full_spec((NB * BW, NB * BW))],
        out_specs=[_row_spec(BW)] * NB,
        out_shape=[jax.ShapeDtypeStruct((N, BW), f32) for _ in range(NB)],
    )(*ss, *gs, dinv, bp, wp)


def _tc_head(ss, gs, dinv, bp, m1w, m1b, m2w, m2b, m3w, m3b):
    return pl.pallas_call(
        _tc_head_body,
        grid=(GRID,),
        in_specs=[_row_spec(BW)] * (2 * NB)
        + [_row_spec(1), _full_spec((1, NB * BW)),
           _full_spec((H, H // 2)), _full_spec((1, H // 2)),
           _full_spec((H // 2, H // 2)), _full_spec((1, H // 2)),
           _full_spec((H // 2, C)), _full_spec((1, C))],
        out_specs=_row_spec(C),
        out_shape=jax.ShapeDtypeStruct((N, C), f32),
    )(*ss, *gs, dinv, bp, m1w, m1b, m2w, m2b, m3w, m3b)


def _pad_w_in(w):
    fi = w.shape[0]
    return jnp.pad(w.reshape(fi, NB, 25), ((0, 0), (0, 0), (0, BW - 25))).reshape(fi, NB * BW)


def _pad_w_both(w):
    w4 = w.reshape(NB, 25, NB, 25)
    w4 = jnp.pad(w4, ((0, 0), (0, BW - 25), (0, 0), (0, BW - 25)))
    return w4.reshape(NB * BW, NB * BW)


def _pad_b(b):
    return jnp.pad(b.reshape(NB, 25), ((0, 0), (0, BW - 25))).reshape(1, NB * BW)


def kernel(x, edge_index, edge_attr, W1, b1, W2, b2, W3, b3,
           M1w, M1b, M2w, M2b, M3w, M3b):
    src = edge_index[0].astype(i32)
    dst = edge_index[1].astype(i32)
    pad = EPAD - E
    srcp = jnp.concatenate([src, jnp.zeros((pad,), i32)])
    dstp = jnp.concatenate([dst, jnp.zeros((pad,), i32)])
    eap = jnp.concatenate([edge_attr.astype(f32), jnp.zeros((pad,), f32)])
    srcp4 = (srcp.reshape(1, EPAD // CH, CH)
             + (jnp.arange(NB, dtype=i32) * N).reshape(NB, 1, 1))
    dstp2 = dstp.reshape(EPAD // CH, CH)
    eap2 = eap.reshape(EPAD // CH, CH)
    zeros1 = jnp.zeros((N,), f32)
    zeros32 = jnp.zeros((N, BW), f32)

    w1p = _pad_w_in(W1)
    w2p = _pad_w_both(W2)
    w3p = _pad_w_both(W3)
    b1p, b2p, b3p = _pad_b(b1), _pad_b(b2), _pad_b(b3)

    dega, degb = _sc_deg()(dstp, eap, zeros1)
    dinv, *g = _tc_first(dega.reshape(N, 1), degb.reshape(N, 1), x, w1p)

    def edge_scatter(g_list):
        s_all = _sc_scatter()(jnp.concatenate(g_list), srcp4, dstp2, eap2,
                              zeros32)
        return [s_all[j * N:(j + 1) * N] for j in range(NB)]

    s = edge_scatter(g)
    g = _tc_mid(s, g, dinv, b1p, w2p)
    s = edge_scatter(g)
    g = _tc_mid(s, g, dinv, b2p, w3p)
    s = edge_scatter(g)

    return _tc_head(s, g, dinv, b3p,
                    M1w, M1b.reshape(1, -1), M2w, M2b.reshape(1, -1),
                    M3w, M3b.reshape(1, -1))


# speedup vs baseline: 6.6392x; 1.0006x over previous
import functools

import jax
import jax.numpy as jnp
from jax import lax
from jax.experimental import pallas as pl
from jax.experimental.pallas import tpu as pltpu
from jax.experimental.pallas import tpu_sc as plsc

N = 50000
E = 800000
DIN = 200
H = 100
C = 11

NB = 4
BW = 32
CH = 128
NS = 16
NC = 2
ROWS_PT = N // NS
EPAD = 802816
DEG_CHUNKS = EPAD // (NC * NS * CH)
SC_CHUNKS = EPAD // (NS * CH)

RB = 2000
GRID = N // RB

f32 = jnp.float32
i32 = jnp.int32


@functools.cache
def _mesh():
    return plsc.VectorSubcoreMesh(core_axis_name="c", subcore_axis_name="s")


def _copy_1d_slices(s, src, stage, dst):
    r0 = pl.multiple_of(s * 3128, 8)

    @pl.when(s < NS - 1)
    def _():
        pltpu.sync_copy(src.at[pl.ds(r0, 3128)], stage.at[pl.ds(0, 3128)])
        pltpu.sync_copy(stage.at[pl.ds(0, 3128)], dst.at[pl.ds(r0, 3128)])

    @pl.when(s == NS - 1)
    def _():
        r1 = 3128 * (NS - 1)
        nr = N - r1
        pltpu.sync_copy(src.at[pl.ds(r1, nr)], stage.at[pl.ds(0, nr)])
        pltpu.sync_copy(stage.at[pl.ds(0, nr)], dst.at[pl.ds(r1, nr)])


def _sc_deg_body(dst_hbm, ea_hbm, z1_hbm, dega, degb, acc, dst_v, ea_v, stage):
    c = lax.axis_index("c")
    s = lax.axis_index("s")
    gtid = c * NS + s
    _copy_1d_slices(s, z1_hbm, stage, acc)
    plsc.subcore_barrier()

    def chunk(i, carry):
        base = pl.multiple_of((gtid * DEG_CHUNKS + i) * CH, CH)
        pltpu.sync_copy(dst_hbm.at[pl.ds(base, CH)], dst_v)
        pltpu.sync_copy(ea_hbm.at[pl.ds(base, CH)], ea_v)
        pltpu.sync_copy(ea_v, acc.at[dst_v], add=True)
        return carry

    lax.fori_loop(0, DEG_CHUNKS, chunk, 0)
    plsc.subcore_barrier()

    @pl.when(c == 0)
    def _():
        _copy_1d_slices(s, acc, stage, dega)

    @pl.when(c == 1)
    def _():
        _copy_1d_slices(s, acc, stage, degb)


STG = 392


def _staged_rows(src, stage, dst, r0, nrows, dst_base=None):
    d0 = r0 if dst_base is None else dst_base
    for off in range(0, nrows, STG):
        n = min(STG, nrows - off)
        ro = pl.multiple_of(r0 + off, 8)
        do = pl.multiple_of(d0 + off, 8)
        pltpu.sync_copy(src.at[pl.ds(ro, n)], stage.at[pl.ds(0, n)])
        pltpu.sync_copy(stage.at[pl.ds(0, n)], dst.at[pl.ds(do, n)])


def _copy_2d_slices(s, src, stage, dst):
    @pl.when(s < NS - 1)
    def _():
        _staged_rows(src, stage, dst, s * 3128, 3128)

    @pl.when(s == NS - 1)
    def _():
        _staged_rows(src, stage, dst, 3128 * (NS - 1), N - 3128 * (NS - 1))


SUP = 8
SUPERS = SC_CHUNKS // SUP


def _sc_scatter_kernel(g_flat, src4_hbm, dst_hbm, ea_hbm, z_hbm, s_all,
                       acc, src_b, dst_b, ea_b, rows_v, stage, sem, sem_s):
    c = lax.axis_index("c")
    s = lax.axis_index("s")

    def drain_scatter():
        pltpu.make_async_copy(rows_v.at[0], acc.at[dst_b.at[0]], sem_s).wait()

    def col_block(jj, carry0):
        j = c * 2 + jj
        _copy_2d_slices(s, z_hbm, stage, acc)
        plsc.subcore_barrier()

        def superchunk(u, carry1):
            @pl.when(u > 0)
            def _():
                drain_scatter()

            row0 = s * SC_CHUNKS + u * SUP
            pltpu.sync_copy(src4_hbm.at[j, pl.ds(row0, SUP)], src_b)
            pltpu.sync_copy(dst_hbm.at[pl.ds(row0, SUP)], dst_b)
            pltpu.sync_copy(ea_hbm.at[pl.ds(row0, SUP)], ea_b)
            pltpu.async_copy(g_flat.at[src_b.at[0]], rows_v.at[0], sem)

            def chunk(k, carry2):
                b = lax.rem(k, 2)

                @pl.when(k > 0)
                def _():
                    drain_scatter()

                @pl.when(k < SUP - 1)
                def _():
                    pltpu.async_copy(g_flat.at[src_b.at[k + 1]],
                                     rows_v.at[lax.rem(k + 1, 2)], sem)

                pltpu.make_async_copy(g_flat.at[src_b.at[0]],
                                      rows_v.at[0], sem).wait()
                for e in range(CH):
                    e_splat = jnp.full((16,), e, i32)
                    k_splat = jnp.full((16,), k, i32)
                    sp = plsc.load_gather(ea_b, [k_splat, e_splat])
                    lo = rows_v[b, e, pl.ds(0, 16)]
                    hi = rows_v[b, e, pl.ds(16, 16)]
                    rows_v[b, e, pl.ds(0, 16)] = lo * sp
                    rows_v[b, e, pl.ds(16, 16)] = hi * sp
                pltpu.make_async_copy(rows_v.at[b], acc.at[dst_b.at[k]],
                                      sem_s).start(add=True)
                return carry2

            lax.fori_loop(0, SUP, chunk, 0)
            return carry1

        lax.fori_loop(0, SUPERS, superchunk, 0)
        drain_scatter()
        plsc.subcore_barrier()

        @pl.when(s < NS - 1)
        def _():
            _staged_rows(acc, stage, s_all, s * 3128, 3128,
                         dst_base=j * N + s * 3128)

        @pl.when(s == NS - 1)
        def _():
            _staged_rows(acc, stage, s_all, 3128 * (NS - 1),
                         N - 3128 * (NS - 1),
                         dst_base=j * N + 3128 * (NS - 1))

        plsc.subcore_barrier()
        return carry0

    lax.fori_loop(0, 2, col_block, 0)


@functools.cache
def _sc_deg():
    return pl.kernel(
        _sc_deg_body,
        out_type=[jax.ShapeDtypeStruct((N,), f32),
                  jax.ShapeDtypeStruct((N,), f32)],
        mesh=_mesh(),
        scratch_types=[
            pltpu.VMEM_SHARED((N,), f32),
            pltpu.VMEM((CH,), i32),
            pltpu.VMEM((CH,), f32),
            pltpu.VMEM((3128,), f32),
        ],
        compiler_params=pltpu.CompilerParams(needs_layout_passes=False,
                                             use_tc_tiling_on_sc=False),
    )


@functools.cache
def _sc_scatter():
    return pl.kernel(
        _sc_scatter_kernel,
        out_type=jax.ShapeDtypeStruct((NB * N, BW), f32),
        mesh=_mesh(),
        scratch_types=[
            pltpu.VMEM_SHARED((N, BW), f32),
            pltpu.VMEM((SUP, CH), i32),
            pltpu.VMEM((SUP, CH), i32),
            pltpu.VMEM((SUP, CH), f32),
            pltpu.VMEM((2, CH, BW), f32),
            pltpu.VMEM((STG, BW), f32),
            pltpu.SemaphoreType.DMA,
            pltpu.SemaphoreType.DMA,
        ],
        compiler_params=pltpu.CompilerParams(needs_layout_passes=False,
                                             use_tc_tiling_on_sc=False),
    )


def _tc_first_body(dega_ref, degb_ref, x_ref, w_ref, dinv_ref,
                   g0_ref, g1_ref, g2_ref, g3_ref):
    deg = dega_ref[...] + degb_ref[...] + 1.0
    dinv = lax.rsqrt(deg)
    dinv_ref[...] = dinv
    hw = jnp.dot(x_ref[...], w_ref[...], preferred_element_type=f32)
    gg = hw * dinv
    g0_ref[...] = gg[:, 0 * BW:1 * BW]
    g1_ref[...] = gg[:, 1 * BW:2 * BW]
    g2_ref[...] = gg[:, 2 * BW:3 * BW]
    g3_ref[...] = gg[:, 3 * BW:4 * BW]


def _tc_mid_body(s0, s1, s2, s3, g0, g1, g2, g3, dinv_ref, b_ref, w_ref,
                 o0, o1, o2, o3):
    dinv = dinv_ref[...]
    hcat = jnp.concatenate(
        [s0[...] + g0[...], s1[...] + g1[...], s2[...] + g2[...], s3[...] + g3[...]],
        axis=1)
    t = jnp.maximum(dinv * hcat + b_ref[...], 0.0)
    hw = jnp.dot(t, w_ref[...], preferred_element_type=f32)
    gg = hw * dinv
    o0[...] = gg[:, 0 * BW:1 * BW]
    o1[...] = gg[:, 1 * BW:2 * BW]
    o2[...] = gg[:, 2 * BW:3 * BW]
    o3[...] = gg[:, 3 * BW:4 * BW]


def _tc_head_body(s0, s1, s2, s3, g0, g1, g2, g3, dinv_ref, b_ref,
                  m1w, m1b, m2w, m2b, m3w, m3b, out_ref):
    dinv = dinv_ref[...]
    hcat = jnp.concatenate(
        [s0[...] + g0[...], s1[...] + g1[...], s2[...] + g2[...], s3[...] + g3[...]],
        axis=1)
    t = jnp.maximum(dinv * hcat + b_ref[...], 0.0)
    h3 = jnp.concatenate([t[:, j * BW:j * BW + 25] for j in range(NB)], axis=1)
    m1 = jnp.maximum(jnp.dot(h3, m1w[...], preferred_element_type=f32) + m1b[...], 0.0)
    m2 = jnp.maximum(jnp.dot(m1, m2w[...], preferred_element_type=f32) + m2b[...], 0.0)
    lg = jnp.dot(m2, m3w[...], preferred_element_type=f32) + m3b[...]
    mx = jnp.max(lg, axis=1, keepdims=True)
    lse = mx + jnp.log(jnp.sum(jnp.exp(lg - mx), axis=1, keepdims=True))
    out_ref[...] = lg - lse


def _row_spec(w):
    return pl.BlockSpec((RB, w), lambda i: (i, 0))


def _full_spec(shape):
    return pl.BlockSpec(shape, lambda i: tuple(0 for _ in shape))


def _tc_first(dega2, degb2, x, w1p):
    return pl.pallas_call(
        _tc_first_body,
        grid=(GRID,),
        in_specs=[_row_spec(1), _row_spec(1), _row_spec(DIN), _full_spec((DIN, NB * BW))],
        out_specs=[_row_spec(1)] + [_row_spec(BW)] * NB,
        out_shape=[jax.ShapeDtypeStruct((N, 1), f32)]
        + [jax.ShapeDtypeStruct((N, BW), f32) for _ in range(NB)],
    )(dega2, degb2, x, w1p)


def _tc_mid(ss, gs, dinv, bp, wp):
    return pl.pallas_call(
        _tc_mid_body,
        grid=(GRID,),
        in_specs=[_row_spec(BW)] * (2 * NB)
        + [_row_spec(1), _full_spec((1, NB * BW)), _full_spec((NB * BW, NB * BW))],
        out_specs=[_row_spec(BW)] * NB,
        out_shape=[jax.ShapeDtypeStruct((N, BW), f32) for _ in range(NB)],
    )(*ss, *gs, dinv, bp, wp)


def _tc_head(ss, gs, dinv, bp, m1w, m1b, m2w, m2b, m3w, m3b):
    return pl.pallas_call(
        _tc_head_body,
        grid=(GRID,),
        in_specs=[_row_spec(BW)] * (2 * NB)
        + [_row_spec(1), _full_spec((1, NB * BW)),
           _full_spec((H, H // 2)), _full_spec((1, H // 2)),
           _full_spec((H // 2, H // 2)), _full_spec((1, H // 2)),
           _full_spec((H // 2, C)), _full_spec((1, C))],
        out_specs=_row_spec(C),
        out_shape=jax.ShapeDtypeStruct((N, C), f32),
    )(*ss, *gs, dinv, bp, m1w, m1b, m2w, m2b, m3w, m3b)


def _pad_w_in(w):
    fi = w.shape[0]
    return jnp.pad(w.reshape(fi, NB, 25), ((0, 0), (0, 0), (0, BW - 25))).reshape(fi, NB * BW)


def _pad_w_both(w):
    w4 = w.reshape(NB, 25, NB, 25)
    w4 = jnp.pad(w4, ((0, 0), (0, BW - 25), (0, 0), (0, BW - 25)))
    return w4.reshape(NB * BW, NB * BW)


def _pad_b(b):
    return jnp.pad(b.reshape(NB, 25), ((0, 0), (0, BW - 25))).reshape(1, NB * BW)


def kernel(x, edge_index, edge_attr, W1, b1, W2, b2, W3, b3,
           M1w, M1b, M2w, M2b, M3w, M3b):
    src = edge_index[0].astype(i32)
    dst = edge_index[1].astype(i32)
    pad = EPAD - E
    srcp = jnp.concatenate([src, jnp.zeros((pad,), i32)])
    dstp = jnp.concatenate([dst, jnp.zeros((pad,), i32)])
    eap = jnp.concatenate([edge_attr.astype(f32), jnp.zeros((pad,), f32)])
    srcp4 = (srcp.reshape(1, EPAD // CH, CH)
             + (jnp.arange(NB, dtype=i32) * N).reshape(NB, 1, 1))
    dstp2 = dstp.reshape(EPAD // CH, CH)
    eap2 = eap.reshape(EPAD // CH, CH)
    zeros1 = jnp.zeros((N,), f32)
    zeros32 = jnp.zeros((N, BW), f32)

    w1p = _pad_w_in(W1)
    w2p = _pad_w_both(W2)
    w3p = _pad_w_both(W3)
    b1p, b2p, b3p = _pad_b(b1), _pad_b(b2), _pad_b(b3)

    dega, degb = _sc_deg()(dstp, eap, zeros1)
    dinv, *g = _tc_first(dega.reshape(N, 1), degb.reshape(N, 1), x, w1p)

    def edge_scatter(g_list):
        s_all = _sc_scatter()(jnp.concatenate(g_list), srcp4, dstp2, eap2,
                              zeros32)
        return [s_all[j * N:(j + 1) * N] for j in range(NB)]

    s = edge_scatter(g)
    g = _tc_mid(s, g, dinv, b1p, w2p)
    s = edge_scatter(g)
    g = _tc_mid(s, g, dinv, b2p, w3p)
    s = edge_scatter(g)

    return _tc_head(s, g, dinv, b3p,
                    M1w, M1b.reshape(1, -1), M2w, M2b.reshape(1, -1),
                    M3w, M3b.reshape(1, -1))


# speedup vs baseline: 7.4825x; 1.1270x over previous
import functools

import jax
import jax.numpy as jnp
from jax import lax
from jax.experimental import pallas as pl
from jax.experimental.pallas import tpu as pltpu
from jax.experimental.pallas import tpu_sc as plsc

N = 50000
E = 800000
DIN = 200
H = 100
C = 11

NB = 4
BW = 32
CH = 128
NS = 16
NC = 2
ROWS_PT = N // NS
EPAD = 802816
DEG_CHUNKS = EPAD // (NC * NS * CH)
SC_CHUNKS = EPAD // (NS * CH)

RB = 2000
GRID = N // RB

f32 = jnp.float32
i32 = jnp.int32


@functools.cache
def _mesh():
    return plsc.VectorSubcoreMesh(core_axis_name="c", subcore_axis_name="s")


def _copy_1d_slices(s, src, stage, dst):
    r0 = pl.multiple_of(s * 3128, 8)

    @pl.when(s < NS - 1)
    def _():
        pltpu.sync_copy(src.at[pl.ds(r0, 3128)], stage.at[pl.ds(0, 3128)])
        pltpu.sync_copy(stage.at[pl.ds(0, 3128)], dst.at[pl.ds(r0, 3128)])

    @pl.when(s == NS - 1)
    def _():
        r1 = 3128 * (NS - 1)
        nr = N - r1
        pltpu.sync_copy(src.at[pl.ds(r1, nr)], stage.at[pl.ds(0, nr)])
        pltpu.sync_copy(stage.at[pl.ds(0, nr)], dst.at[pl.ds(r1, nr)])


def _sc_deg_body(dst_hbm, ea_hbm, z1_hbm, dega, degb, acc, dst_v, ea_v, stage):
    c = lax.axis_index("c")
    s = lax.axis_index("s")
    gtid = c * NS + s
    _copy_1d_slices(s, z1_hbm, stage, acc)
    plsc.subcore_barrier()

    def chunk(i, carry):
        base = pl.multiple_of((gtid * DEG_CHUNKS + i) * CH, CH)
        pltpu.sync_copy(dst_hbm.at[pl.ds(base, CH)], dst_v)
        pltpu.sync_copy(ea_hbm.at[pl.ds(base, CH)], ea_v)
        pltpu.sync_copy(ea_v, acc.at[dst_v], add=True)
        return carry

    lax.fori_loop(0, DEG_CHUNKS, chunk, 0)
    plsc.subcore_barrier()

    @pl.when(c == 0)
    def _():
        _copy_1d_slices(s, acc, stage, dega)

    @pl.when(c == 1)
    def _():
        _copy_1d_slices(s, acc, stage, degb)


STG = 256


def _staged_rows(src, stage, dst, r0, nrows, dst_base=None):
    d0 = r0 if dst_base is None else dst_base
    for off in range(0, nrows, STG):
        n = min(STG, nrows - off)
        ro = pl.multiple_of(r0 + off, 8)
        do = pl.multiple_of(d0 + off, 8)
        pltpu.sync_copy(src.at[pl.ds(ro, n)], stage.at[pl.ds(0, n)])
        pltpu.sync_copy(stage.at[pl.ds(0, n)], dst.at[pl.ds(do, n)])


def _copy_2d_slices(s, src, stage, dst):
    @pl.when(s < NS - 1)
    def _():
        _staged_rows(src, stage, dst, s * 3128, 3128)

    @pl.when(s == NS - 1)
    def _():
        _staged_rows(src, stage, dst, 3128 * (NS - 1), N - 3128 * (NS - 1))


SUP = 8
SUPERS = SC_CHUNKS // SUP
DEPTH = 4


def _sc_scatter_kernel(g_flat, pk_hbm, z_hbm, s_all,
                       acc, pk, src_sh, rows_v, stage, sem, sem_s):
    c = lax.axis_index("c")
    s = lax.axis_index("s")

    def drain_scatter():
        pltpu.make_async_copy(rows_v.at[0], acc.at[pk.at[0, 1]], sem_s).wait()

    def col_block(jj, carry0):
        j = c * 2 + jj
        jn = j * N
        _copy_2d_slices(s, z_hbm, stage, acc)
        plsc.subcore_barrier()

        def shift_issue(kt):
            b2 = lax.rem(kt, DEPTH)
            for gg in range(CH // 16):
                sl = pl.ds(gg * 16, 16)
                src_sh[b2, sl] = pk[kt, 0, sl] + jn
            pltpu.async_copy(g_flat.at[src_sh.at[b2]],
                             rows_v.at[lax.rem(kt, DEPTH)], sem)

        def superchunk(u, carry1):
            @pl.when(u > 0)
            def _():
                drain_scatter()
                drain_scatter()

            row0 = s * SC_CHUNKS + u * SUP
            pltpu.sync_copy(pk_hbm.at[pl.ds(row0, SUP)], pk)
            shift_issue(0)
            shift_issue(1)

            def chunk(k, carry2):
                b = lax.rem(k, DEPTH)

                @pl.when(k >= 2)
                def _():
                    drain_scatter()

                @pl.when(k < SUP - 2)
                def _():
                    shift_issue(k + 2)

                pltpu.make_async_copy(g_flat.at[src_sh.at[0]],
                                      rows_v.at[0], sem).wait()
                for e in range(CH):
                    e_splat = jnp.full((16,), e, i32)
                    k_splat = jnp.full((16,), k, i32)
                    f_splat = jnp.full((16,), 2, i32)
                    sp = plsc.bitcast(
                        plsc.load_gather(pk, [k_splat, f_splat, e_splat]), f32)
                    lo = rows_v[b, e, pl.ds(0, 16)]
                    hi = rows_v[b, e, pl.ds(16, 16)]
                    rows_v[b, e, pl.ds(0, 16)] = lo * sp
                    rows_v[b, e, pl.ds(16, 16)] = hi * sp
                pltpu.make_async_copy(rows_v.at[b], acc.at[pk.at[k, 1]],
                                      sem_s).start(add=True)
                return carry2

            lax.fori_loop(0, SUP, chunk, 0)
            return carry1

        lax.fori_loop(0, SUPERS, superchunk, 0)
        drain_scatter()
        drain_scatter()
        plsc.subcore_barrier()

        @pl.when(s < NS - 1)
        def _():
            _staged_rows(acc, stage, s_all, s * 3128, 3128,
                         dst_base=j * N + s * 3128)

        @pl.when(s == NS - 1)
        def _():
            _staged_rows(acc, stage, s_all, 3128 * (NS - 1),
                         N - 3128 * (NS - 1),
                         dst_base=j * N + 3128 * (NS - 1))

        plsc.subcore_barrier()
        return carry0

    lax.fori_loop(0, 2, col_block, 0)


@functools.cache
def _sc_deg():
    return pl.kernel(
        _sc_deg_body,
        out_type=[jax.ShapeDtypeStruct((N,), f32),
                  jax.ShapeDtypeStruct((N,), f32)],
        mesh=_mesh(),
        scratch_types=[
            pltpu.VMEM_SHARED((N,), f32),
            pltpu.VMEM((CH,), i32),
            pltpu.VMEM((CH,), f32),
            pltpu.VMEM((3128,), f32),
        ],
        compiler_params=pltpu.CompilerParams(needs_layout_passes=False,
                                             use_tc_tiling_on_sc=False),
    )


@functools.cache
def _sc_scatter():
    return pl.kernel(
        _sc_scatter_kernel,
        out_type=jax.ShapeDtypeStruct((NB * N, BW), f32),
        mesh=_mesh(),
        scratch_types=[
            pltpu.VMEM_SHARED((N, BW), f32),
            pltpu.VMEM((SUP, 3, CH), i32),
            pltpu.VMEM((DEPTH, CH), i32),
            pltpu.VMEM((DEPTH, CH, BW), f32),
            pltpu.VMEM((STG, BW), f32),
            pltpu.SemaphoreType.DMA,
            pltpu.SemaphoreType.DMA,
        ],
        compiler_params=pltpu.CompilerParams(needs_layout_passes=False,
                                             use_tc_tiling_on_sc=False),
    )


def _tc_first_body(dega_ref, degb_ref, x_ref, w_ref, dinv_ref,
                   g0_ref, g1_ref, g2_ref, g3_ref):
    deg = dega_ref[...] + degb_ref[...] + 1.0
    dinv = lax.rsqrt(deg)
    dinv_ref[...] = dinv
    hw = jnp.dot(x_ref[...], w_ref[...], preferred_element_type=f32)
    gg = hw * dinv
    g0_ref[...] = gg[:, 0 * BW:1 * BW]
    g1_ref[...] = gg[:, 1 * BW:2 * BW]
    g2_ref[...] = gg[:, 2 * BW:3 * BW]
    g3_ref[...] = gg[:, 3 * BW:4 * BW]


def _tc_mid_body(s0, s1, s2, s3, g0, g1, g2, g3, dinv_ref, b_ref, w_ref,
                 o0, o1, o2, o3):
    dinv = dinv_ref[...]
    hcat = jnp.concatenate(
        [s0[...] + g0[...], s1[...] + g1[...], s2[...] + g2[...], s3[...] + g3[...]],
        axis=1)
    t = jnp.maximum(dinv * hcat + b_ref[...], 0.0)
    hw = jnp.dot(t, w_ref[...], preferred_element_type=f32)
    gg = hw * dinv
    o0[...] = gg[:, 0 * BW:1 * BW]
    o1[...] = gg[:, 1 * BW:2 * BW]
    o2[...] = gg[:, 2 * BW:3 * BW]
    o3[...] = gg[:, 3 * BW:4 * BW]


def _tc_head_body(s0, s1, s2, s3, g0, g1, g2, g3, dinv_ref, b_ref,
                  m1w, m1b, m2w, m2b, m3w, m3b, out_ref):
    dinv = dinv_ref[...]
    hcat = jnp.concatenate(
        [s0[...] + g0[...], s1[...] + g1[...], s2[...] + g2[...], s3[...] + g3[...]],
        axis=1)
    t = jnp.maximum(dinv * hcat + b_ref[...], 0.0)
    h3 = jnp.concatenate([t[:, j * BW:j * BW + 25] for j in range(NB)], axis=1)
    m1 = jnp.maximum(jnp.dot(h3, m1w[...], preferred_element_type=f32) + m1b[...], 0.0)
    m2 = jnp.maximum(jnp.dot(m1, m2w[...], preferred_element_type=f32) + m2b[...], 0.0)
    lg = jnp.dot(m2, m3w[...], preferred_element_type=f32) + m3b[...]
    mx = jnp.max(lg, axis=1, keepdims=True)
    lse = mx + jnp.log(jnp.sum(jnp.exp(lg - mx), axis=1, keepdims=True))
    out_ref[...] = lg - lse


def _row_spec(w):
    return pl.BlockSpec((RB, w), lambda i: (i, 0))


def _full_spec(shape):
    return pl.BlockSpec(shape, lambda i: tuple(0 for _ in shape))


def _tc_first(dega2, degb2, x, w1p):
    return pl.pallas_call(
        _tc_first_body,
        grid=(GRID,),
        in_specs=[_row_spec(1), _row_spec(1), _row_spec(DIN), _full_spec((DIN, NB * BW))],
        out_specs=[_row_spec(1)] + [_row_spec(BW)] * NB,
        out_shape=[jax.ShapeDtypeStruct((N, 1), f32)]
        + [jax.ShapeDtypeStruct((N, BW), f32) for _ in range(NB)],
    )(dega2, degb2, x, w1p)


def _tc_mid(ss, gs, dinv, bp, wp):
    return pl.pallas_call(
        _tc_mid_body,
        grid=(GRID,),
        in_specs=[_row_spec(BW)] * (2 * NB)
        + [_row_spec(1), _full_spec((1, NB * BW)), _full_spec((NB * BW, NB * BW))],
        out_specs=[_row_spec(BW)] * NB,
        out_shape=[jax.ShapeDtypeStruct((N, BW), f32) for _ in range(NB)],
    )(*ss, *gs, dinv, bp, wp)


def _tc_head(ss, gs, dinv, bp, m1w, m1b, m2w, m2b, m3w, m3b):
    return pl.pallas_call(
        _tc_head_body,
        grid=(GRID,),
        in_specs=[_row_spec(BW)] * (2 * NB)
        + [_row_spec(1), _full_spec((1, NB * BW)),
           _full_spec((H, H // 2)), _full_spec((1, H // 2)),
           _full_spec((H // 2, H // 2)), _full_spec((1, H // 2)),
           _full_spec((H // 2, C)), _full_spec((1, C))],
        out_specs=_row_spec(C),
        out_shape=jax.ShapeDtypeStruct((N, C), f32),
    )(*ss, *gs, dinv, bp, m1w, m1b, m2w, m2b, m3w, m3b)


def _pad_w_in(w):
    fi = w.shape[0]
    return jnp.pad(w.reshape(fi, NB, 25), ((0, 0), (0, 0), (0, BW - 25))).reshape(fi, NB * BW)


def _pad_w_both(w):
    w4 = w.reshape(NB, 25, NB, 25)
    w4 = jnp.pad(w4, ((0, 0), (0, BW - 25), (0, 0), (0, BW - 25)))
    return w4.reshape(NB * BW, NB * BW)


def _pad_b(b):
    return jnp.pad(b.reshape(NB, 25), ((0, 0), (0, BW - 25))).reshape(1, NB * BW)


def kernel(x, edge_index, edge_attr, W1, b1, W2, b2, W3, b3,
           M1w, M1b, M2w, M2b, M3w, M3b):
    src = edge_index[0].astype(i32)
    dst = edge_index[1].astype(i32)
    pad = EPAD - E
    srcp = jnp.concatenate([src, jnp.zeros((pad,), i32)])
    dstp = jnp.concatenate([dst, jnp.zeros((pad,), i32)])
    eap = jnp.concatenate([edge_attr.astype(f32), jnp.zeros((pad,), f32)])
    pk = jnp.stack([srcp.reshape(EPAD // CH, CH),
                    dstp.reshape(EPAD // CH, CH),
                    lax.bitcast_convert_type(eap, i32).reshape(EPAD // CH, CH)],
                   axis=1)
    zeros1 = jnp.zeros((N,), f32)
    zeros32 = jnp.zeros((N, BW), f32)

    w1p = _pad_w_in(W1)
    w2p = _pad_w_both(W2)
    w3p = _pad_w_both(W3)
    b1p, b2p, b3p = _pad_b(b1), _pad_b(b2), _pad_b(b3)

    dega, degb = _sc_deg()(dstp, eap, zeros1)
    dinv, *g = _tc_first(dega.reshape(N, 1), degb.reshape(N, 1), x, w1p)

    def edge_scatter(g_list):
        s_all = _sc_scatter()(jnp.concatenate(g_list), pk, zeros32)
        return [s_all[j * N:(j + 1) * N] for j in range(NB)]

    s = edge_scatter(g)
    g = _tc_mid(s, g, dinv, b1p, w2p)
    s = edge_scatter(g)
    g = _tc_mid(s, g, dinv, b2p, w3p)
    s = edge_scatter(g)

    return _tc_head(s, g, dinv, b3p,
                    M1w, M1b.reshape(1, -1), M2w, M2b.reshape(1, -1),
                    M3w, M3b.reshape(1, -1))


# speedup vs baseline: 9.7367x; 1.3013x over previous
import functools

import jax
import jax.numpy as jnp
from jax import lax
from jax.experimental import pallas as pl
from jax.experimental.pallas import tpu as pltpu
from jax.experimental.pallas import tpu_sc as plsc

N = 50000
E = 800000
DIN = 200
H = 100
C = 11

NB = 4
BW = 32
CH = 128
NS = 16
NC = 2
ROWS_PT = N // NS
EPAD = 802816
DEG_CHUNKS = EPAD // (NC * NS * CH)
SC_CHUNKS = EPAD // (NS * CH)

RB = 2000
GRID = N // RB

f32 = jnp.float32
i32 = jnp.int32


@functools.cache
def _mesh():
    return plsc.VectorSubcoreMesh(core_axis_name="c", subcore_axis_name="s")


def _copy_1d_slices(s, src, stage, dst):
    r0 = pl.multiple_of(s * 3128, 8)

    @pl.when(s < NS - 1)
    def _():
        pltpu.sync_copy(src.at[pl.ds(r0, 3128)], stage.at[pl.ds(0, 3128)])
        pltpu.sync_copy(stage.at[pl.ds(0, 3128)], dst.at[pl.ds(r0, 3128)])

    @pl.when(s == NS - 1)
    def _():
        r1 = 3128 * (NS - 1)
        nr = N - r1
        pltpu.sync_copy(src.at[pl.ds(r1, nr)], stage.at[pl.ds(0, nr)])
        pltpu.sync_copy(stage.at[pl.ds(0, nr)], dst.at[pl.ds(r1, nr)])


def _sc_deg_body(dst_hbm, ea_hbm, z1_hbm, dega, degb, acc, dst_v, ea_v, stage):
    c = lax.axis_index("c")
    s = lax.axis_index("s")
    gtid = c * NS + s
    _copy_1d_slices(s, z1_hbm, stage, acc)
    plsc.subcore_barrier()

    def chunk(i, carry):
        base = pl.multiple_of((gtid * DEG_CHUNKS + i) * CH, CH)
        pltpu.sync_copy(dst_hbm.at[pl.ds(base, CH)], dst_v)
        pltpu.sync_copy(ea_hbm.at[pl.ds(base, CH)], ea_v)
        pltpu.sync_copy(ea_v, acc.at[dst_v], add=True)
        return carry

    lax.fori_loop(0, DEG_CHUNKS, chunk, 0)
    plsc.subcore_barrier()

    @pl.when(c == 0)
    def _():
        _copy_1d_slices(s, acc, stage, dega)

    @pl.when(c == 1)
    def _():
        _copy_1d_slices(s, acc, stage, degb)


STG = 256


def _staged_rows(src, stage, dst, r0, nrows, dst_base=None):
    d0 = r0 if dst_base is None else dst_base
    for off in range(0, nrows, STG):
        n = min(STG, nrows - off)
        ro = pl.multiple_of(r0 + off, 8)
        do = pl.multiple_of(d0 + off, 8)
        pltpu.sync_copy(src.at[pl.ds(ro, n)], stage.at[pl.ds(0, n)])
        pltpu.sync_copy(stage.at[pl.ds(0, n)], dst.at[pl.ds(do, n)])


def _copy_2d_slices(s, src, stage, dst):
    @pl.when(s < NS - 1)
    def _():
        _staged_rows(src, stage, dst, s * 3128, 3128)

    @pl.when(s == NS - 1)
    def _():
        _staged_rows(src, stage, dst, 3128 * (NS - 1), N - 3128 * (NS - 1))


SUP = 14
SUPERS = SC_CHUNKS // SUP
DEPTH = 4


def _lane_splat(vec16, lane):
    idx = jnp.full((16, 1), lane, i32)
    return lax.gather(
        vec16, idx,
        dimension_numbers=lax.GatherDimensionNumbers(
            offset_dims=(), collapsed_slice_dims=(0,), start_index_map=(0,)),
        slice_sizes=(1,),
        mode=lax.GatherScatterMode.PROMISE_IN_BOUNDS)


def _sc_scatter_kernel(g_flat, pk_hbm, z_hbm, s_all,
                       acc, pk, src_sh, rows_v, stage, sem, sem_s):
    c = lax.axis_index("c")
    s = lax.axis_index("s")

    def drain_scatter():
        pltpu.make_async_copy(rows_v.at[0], acc.at[pk.at[0, 1]], sem_s).wait()

    def col_block(jj, carry0):
        j = c * 2 + jj
        jn = j * N
        _copy_2d_slices(s, z_hbm, stage, acc)
        plsc.subcore_barrier()

        def shift_issue(kt):
            b2 = lax.rem(kt, DEPTH)
            for gg in range(CH // 16):
                sl = pl.ds(gg * 16, 16)
                src_sh[b2, sl] = pk[kt, 0, sl] + jn
            pltpu.async_copy(g_flat.at[src_sh.at[b2]],
                             rows_v.at[lax.rem(kt, DEPTH)], sem)

        def superchunk(u, carry1):
            @pl.when(u > 0)
            def _():
                drain_scatter()
                drain_scatter()

            row0 = s * SC_CHUNKS + u * SUP
            pltpu.sync_copy(pk_hbm.at[pl.ds(row0, SUP)], pk)
            shift_issue(0)
            shift_issue(1)

            def chunk(k, carry2):
                b = lax.rem(k, DEPTH)

                @pl.when(k >= 2)
                def _():
                    drain_scatter()

                @pl.when(k < SUP - 2)
                def _():
                    shift_issue(k + 2)

                pltpu.make_async_copy(g_flat.at[src_sh.at[0]],
                                      rows_v.at[0], sem).wait()
                for e in range(CH):
                    if e % 16 == 0:
                        ea16 = plsc.bitcast(pk[k, 2, pl.ds(e, 16)], f32)
                    sp = _lane_splat(ea16, e % 16)
                    lo = rows_v[b, e, pl.ds(0, 16)]
                    hi = rows_v[b, e, pl.ds(16, 16)]
                    rows_v[b, e, pl.ds(0, 16)] = lo * sp
                    rows_v[b, e, pl.ds(16, 16)] = hi * sp
                pltpu.make_async_copy(rows_v.at[b], acc.at[pk.at[k, 1]],
                                      sem_s).start(add=True)
                return carry2

            lax.fori_loop(0, SUP, chunk, 0)
            return carry1

        lax.fori_loop(0, SUPERS, superchunk, 0)
        drain_scatter()
        drain_scatter()
        plsc.subcore_barrier()

        @pl.when(s < NS - 1)
        def _():
            _staged_rows(acc, stage, s_all, s * 3128, 3128,
                         dst_base=j * N + s * 3128)

        @pl.when(s == NS - 1)
        def _():
            _staged_rows(acc, stage, s_all, 3128 * (NS - 1),
                         N - 3128 * (NS - 1),
                         dst_base=j * N + 3128 * (NS - 1))

        plsc.subcore_barrier()
        return carry0

    lax.fori_loop(0, 2, col_block, 0)


@functools.cache
def _sc_deg():
    return pl.kernel(
        _sc_deg_body,
        out_type=[jax.ShapeDtypeStruct((N,), f32),
                  jax.ShapeDtypeStruct((N,), f32)],
        mesh=_mesh(),
        scratch_types=[
            pltpu.VMEM_SHARED((N,), f32),
            pltpu.VMEM((CH,), i32),
            pltpu.VMEM((CH,), f32),
            pltpu.VMEM((3128,), f32),
        ],
        compiler_params=pltpu.CompilerParams(needs_layout_passes=False,
                                             use_tc_tiling_on_sc=False),
    )


@functools.cache
def _sc_scatter():
    return pl.kernel(
        _sc_scatter_kernel,
        out_type=jax.ShapeDtypeStruct((NB * N, BW), f32),
        mesh=_mesh(),
        scratch_types=[
            pltpu.VMEM_SHARED((N, BW), f32),
            pltpu.VMEM((SUP, 3, CH), i32),
            pltpu.VMEM((DEPTH, CH), i32),
            pltpu.VMEM((DEPTH, CH, BW), f32),
            pltpu.VMEM((STG, BW), f32),
            pltpu.SemaphoreType.DMA,
            pltpu.SemaphoreType.DMA,
        ],
        compiler_params=pltpu.CompilerParams(needs_layout_passes=False,
                                             use_tc_tiling_on_sc=False),
    )


def _tc_first_body(dega_ref, degb_ref, x_ref, w_ref, dinv_ref,
                   g0_ref, g1_ref, g2_ref, g3_ref):
    deg = dega_ref[...] + degb_ref[...] + 1.0
    dinv = lax.rsqrt(deg)
    dinv_ref[...] = dinv
    hw = jnp.dot(x_ref[...], w_ref[...], preferred_element_type=f32)
    gg = hw * dinv
    g0_ref[...] = gg[:, 0 * BW:1 * BW]
    g1_ref[...] = gg[:, 1 * BW:2 * BW]
    g2_ref[...] = gg[:, 2 * BW:3 * BW]
    g3_ref[...] = gg[:, 3 * BW:4 * BW]


def _tc_mid_body(s0, s1, s2, s3, g0, g1, g2, g3, dinv_ref, b_ref, w_ref,
                 o0, o1, o2, o3):
    dinv = dinv_ref[...]
    hcat = jnp.concatenate(
        [s0[...] + g0[...], s1[...] + g1[...], s2[...] + g2[...], s3[...] + g3[...]],
        axis=1)
    t = jnp.maximum(dinv * hcat + b_ref[...], 0.0)
    hw = jnp.dot(t, w_ref[...], preferred_element_type=f32)
    gg = hw * dinv
    o0[...] = gg[:, 0 * BW:1 * BW]
    o1[...] = gg[:, 1 * BW:2 * BW]
    o2[...] = gg[:, 2 * BW:3 * BW]
    o3[...] = gg[:, 3 * BW:4 * BW]


def _tc_head_body(s0, s1, s2, s3, g0, g1, g2, g3, dinv_ref, b_ref,
                  m1w, m1b, m2w, m2b, m3w, m3b, out_ref):
    dinv = dinv_ref[...]
    hcat = jnp.concatenate(
        [s0[...] + g0[...], s1[...] + g1[...], s2[...] + g2[...], s3[...] + g3[...]],
        axis=1)
    t = jnp.maximum(dinv * hcat + b_ref[...], 0.0)
    h3 = jnp.concatenate([t[:, j * BW:j * BW + 25] for j in range(NB)], axis=1)
    m1 = jnp.maximum(jnp.dot(h3, m1w[...], preferred_element_type=f32) + m1b[...], 0.0)
    m2 = jnp.maximum(jnp.dot(m1, m2w[...], preferred_element_type=f32) + m2b[...], 0.0)
    lg = jnp.dot(m2, m3w[...], preferred_element_type=f32) + m3b[...]
    mx = jnp.max(lg, axis=1, keepdims=True)
    lse = mx + jnp.log(jnp.sum(jnp.exp(lg - mx), axis=1, keepdims=True))
    out_ref[...] = lg - lse


def _row_spec(w):
    return pl.BlockSpec((RB, w), lambda i: (i, 0))


def _full_spec(shape):
    return pl.BlockSpec(shape, lambda i: tuple(0 for _ in shape))


def _tc_first(dega2, degb2, x, w1p):
    return pl.pallas_call(
        _tc_first_body,
        grid=(GRID,),
        in_specs=[_row_spec(1), _row_spec(1), _row_spec(DIN), _full_spec((DIN, NB * BW))],
        out_specs=[_row_spec(1)] + [_row_spec(BW)] * NB,
        out_shape=[jax.ShapeDtypeStruct((N, 1), f32)]
        + [jax.ShapeDtypeStruct((N, BW), f32) for _ in range(NB)],
    )(dega2, degb2, x, w1p)


def _tc_mid(ss, gs, dinv, bp, wp):
    return pl.pallas_call(
        _tc_mid_body,
        grid=(GRID,),
        in_specs=[_row_spec(BW)] * (2 * NB)
        + [_row_spec(1), _full_spec((1, NB * BW)), _full_spec((NB * BW, NB * BW))],
        out_specs=[_row_spec(BW)] * NB,
        out_shape=[jax.ShapeDtypeStruct((N, BW), f32) for _ in range(NB)],
    )(*ss, *gs, dinv, bp, wp)


def _tc_head(ss, gs, dinv, bp, m1w, m1b, m2w, m2b, m3w, m3b):
    return pl.pallas_call(
        _tc_head_body,
        grid=(GRID,),
        in_specs=[_row_spec(BW)] * (2 * NB)
        + [_row_spec(1), _full_spec((1, NB * BW)),
           _full_spec((H, H // 2)), _full_spec((1, H // 2)),
           _full_spec((H // 2, H // 2)), _full_spec((1, H // 2)),
           _full_spec((H // 2, C)), _full_spec((1, C))],
        out_specs=_row_spec(C),
        out_shape=jax.ShapeDtypeStruct((N, C), f32),
    )(*ss, *gs, dinv, bp, m1w, m1b, m2w, m2b, m3w, m3b)


def _pad_w_in(w):
    fi = w.shape[0]
    return jnp.pad(w.reshape(fi, NB, 25), ((0, 0), (0, 0), (0, BW - 25))).reshape(fi, NB * BW)


def _pad_w_both(w):
    w4 = w.reshape(NB, 25, NB, 25)
    w4 = jnp.pad(w4, ((0, 0), (0, BW - 25), (0, 0), (0, BW - 25)))
    return w4.reshape(NB * BW, NB * BW)


def _pad_b(b):
    return jnp.pad(b.reshape(NB, 25), ((0, 0), (0, BW - 25))).reshape(1, NB * BW)


def kernel(x, edge_index, edge_attr, W1, b1, W2, b2, W3, b3,
           M1w, M1b, M2w, M2b, M3w, M3b):
    src = edge_index[0].astype(i32)
    dst = edge_index[1].astype(i32)
    pad = EPAD - E
    srcp = jnp.concatenate([src, jnp.zeros((pad,), i32)])
    dstp = jnp.concatenate([dst, jnp.zeros((pad,), i32)])
    eap = jnp.concatenate([edge_attr.astype(f32), jnp.zeros((pad,), f32)])
    pk = jnp.stack([srcp.reshape(EPAD // CH, CH),
                    dstp.reshape(EPAD // CH, CH),
                    lax.bitcast_convert_type(eap, i32).reshape(EPAD // CH, CH)],
                   axis=1)
    zeros1 = jnp.zeros((N,), f32)
    zeros32 = jnp.zeros((N, BW), f32)

    w1p = _pad_w_in(W1)
    w2p = _pad_w_both(W2)
    w3p = _pad_w_both(W3)
    b1p, b2p, b3p = _pad_b(b1), _pad_b(b2), _pad_b(b3)

    dega, degb = _sc_deg()(dstp, eap, zeros1)
    dinv, *g = _tc_first(dega.reshape(N, 1), degb.reshape(N, 1), x, w1p)

    def edge_scatter(g_list):
        s_all = _sc_scatter()(jnp.concatenate(g_list), pk, zeros32)
        return [s_all[j * N:(j + 1) * N] for j in range(NB)]

    s = edge_scatter(g)
    g = _tc_mid(s, g, dinv, b1p, w2p)
    s = edge_scatter(g)
    g = _tc_mid(s, g, dinv, b2p, w3p)
    s = edge_scatter(g)

    return _tc_head(s, g, dinv, b3p,
                    M1w, M1b.reshape(1, -1), M2w, M2b.reshape(1, -1),
                    M3w, M3b.reshape(1, -1))


# speedup vs baseline: 9.7396x; 1.0003x over previous
import functools

import jax
import jax.numpy as jnp
from jax import lax
from jax.experimental import pallas as pl
from jax.experimental.pallas import tpu as pltpu
from jax.experimental.pallas import tpu_sc as plsc

N = 50000
E = 800000
DIN = 200
H = 100
C = 11

NB = 4
BW = 32
CH = 128
NS = 16
NC = 2
ROWS_PT = N // NS
EPAD = 802816
DEG_CHUNKS = EPAD // (NC * NS * CH)
SC_CHUNKS = EPAD // (NS * CH)

RB = 2000
GRID = N // RB

f32 = jnp.float32
i32 = jnp.int32


@functools.cache
def _mesh():
    return plsc.VectorSubcoreMesh(core_axis_name="c", subcore_axis_name="s")


def _copy_1d_slices(s, src, stage, dst):
    r0 = pl.multiple_of(s * 3128, 8)

    @pl.when(s < NS - 1)
    def _():
        pltpu.sync_copy(src.at[pl.ds(r0, 3128)], stage.at[pl.ds(0, 3128)])
        pltpu.sync_copy(stage.at[pl.ds(0, 3128)], dst.at[pl.ds(r0, 3128)])

    @pl.when(s == NS - 1)
    def _():
        r1 = 3128 * (NS - 1)
        nr = N - r1
        pltpu.sync_copy(src.at[pl.ds(r1, nr)], stage.at[pl.ds(0, nr)])
        pltpu.sync_copy(stage.at[pl.ds(0, nr)], dst.at[pl.ds(r1, nr)])


def _sc_deg_body(dst_hbm, ea_hbm, z1_hbm, dega, degb, acc, dst_v, ea_v, stage):
    c = lax.axis_index("c")
    s = lax.axis_index("s")
    gtid = c * NS + s
    _copy_1d_slices(s, z1_hbm, stage, acc)
    plsc.subcore_barrier()

    def chunk(i, carry):
        base = pl.multiple_of((gtid * DEG_CHUNKS + i) * CH, CH)
        pltpu.sync_copy(dst_hbm.at[pl.ds(base, CH)], dst_v)
        pltpu.sync_copy(ea_hbm.at[pl.ds(base, CH)], ea_v)
        pltpu.sync_copy(ea_v, acc.at[dst_v], add=True)
        return carry

    lax.fori_loop(0, DEG_CHUNKS, chunk, 0)
    plsc.subcore_barrier()

    @pl.when(c == 0)
    def _():
        _copy_1d_slices(s, acc, stage, dega)

    @pl.when(c == 1)
    def _():
        _copy_1d_slices(s, acc, stage, degb)


STG = 256


def _staged_rows(src, stage, dst, r0, nrows, dst_base=None):
    d0 = r0 if dst_base is None else dst_base
    for off in range(0, nrows, STG):
        n = min(STG, nrows - off)
        ro = pl.multiple_of(r0 + off, 8)
        do = pl.multiple_of(d0 + off, 8)
        pltpu.sync_copy(src.at[pl.ds(ro, n)], stage.at[pl.ds(0, n)])
        pltpu.sync_copy(stage.at[pl.ds(0, n)], dst.at[pl.ds(do, n)])


def _copy_2d_slices(s, src, stage, dst):
    @pl.when(s < NS - 1)
    def _():
        _staged_rows(src, stage, dst, s * 3128, 3128)

    @pl.when(s == NS - 1)
    def _():
        _staged_rows(src, stage, dst, 3128 * (NS - 1), N - 3128 * (NS - 1))


SUP = 14
SUPERS = SC_CHUNKS // SUP
DEPTH = 4


def _lane_splat(vec16, lane):
    idx = jnp.full((16, 1), lane, i32)
    return lax.gather(
        vec16, idx,
        dimension_numbers=lax.GatherDimensionNumbers(
            offset_dims=(), collapsed_slice_dims=(0,), start_index_map=(0,)),
        slice_sizes=(1,),
        mode=lax.GatherScatterMode.PROMISE_IN_BOUNDS)


def _sc_scatter_kernel(g_flat, pk_hbm, z_hbm, s_all,
                       acc, pk, src_sh, rows_v, stage, sem, sem_s):
    c = lax.axis_index("c")
    s = lax.axis_index("s")

    def drain_scatter():
        pltpu.make_async_copy(rows_v.at[0], acc.at[pk.at[0, 1]], sem_s).wait()

    def col_block(jj, carry0):
        j = c * 2 + jj
        jn = j * N
        _copy_2d_slices(s, z_hbm, stage, acc)
        plsc.subcore_barrier()

        def shift_issue(kt):
            b2 = lax.rem(kt, DEPTH)
            for gg in range(CH // 16):
                sl = pl.ds(gg * 16, 16)
                src_sh[b2, sl] = pk[kt, 0, sl] + jn
            pltpu.async_copy(g_flat.at[src_sh.at[b2]],
                             rows_v.at[lax.rem(kt, DEPTH)], sem)

        def superchunk(u, carry1):
            @pl.when(u > 0)
            def _():
                drain_scatter()
                drain_scatter()

            row0 = s * SC_CHUNKS + u * SUP
            pltpu.sync_copy(pk_hbm.at[pl.ds(row0, SUP)], pk)
            shift_issue(0)
            shift_issue(1)

            def chunk(k, carry2):
                b = lax.rem(k, DEPTH)

                @pl.when(k >= 2)
                def _():
                    drain_scatter()

                @pl.when(k < SUP - 2)
                def _():
                    shift_issue(k + 2)

                pltpu.make_async_copy(g_flat.at[src_sh.at[0]],
                                      rows_v.at[0], sem).wait()
                for e in range(CH):
                    if e % 16 == 0:
                        ea16 = plsc.bitcast(pk[k, 2, pl.ds(e, 16)], f32)
                    sp = _lane_splat(ea16, e % 16)
                    lo = rows_v[b, e, pl.ds(0, 16)]
                    hi = rows_v[b, e, pl.ds(16, 16)]
                    rows_v[b, e, pl.ds(0, 16)] = lo * sp
                    rows_v[b, e, pl.ds(16, 16)] = hi * sp
                pltpu.make_async_copy(rows_v.at[b], acc.at[pk.at[k, 1]],
                                      sem_s).start(add=True)
                return carry2

            lax.fori_loop(0, SUP, chunk, 0)
            return carry1

        lax.fori_loop(0, SUPERS, superchunk, 0)
        drain_scatter()
        drain_scatter()
        plsc.subcore_barrier()

        @pl.when(s < NS - 1)
        def _():
            _staged_rows(acc, stage, s_all, s * 3128, 3128,
                         dst_base=j * N + s * 3128)

        @pl.when(s == NS - 1)
        def _():
            _staged_rows(acc, stage, s_all, 3128 * (NS - 1),
                         N - 3128 * (NS - 1),
                         dst_base=j * N + 3128 * (NS - 1))

        plsc.subcore_barrier()
        return carry0

    lax.fori_loop(0, 2, col_block, 0)


@functools.cache
def _sc_deg():
    return pl.kernel(
        _sc_deg_body,
        out_type=[jax.ShapeDtypeStruct((N,), f32),
                  jax.ShapeDtypeStruct((N,), f32)],
        mesh=_mesh(),
        scratch_types=[
            pltpu.VMEM_SHARED((N,), f32),
            pltpu.VMEM((CH,), i32),
            pltpu.VMEM((CH,), f32),
            pltpu.VMEM((3128,), f32),
        ],
        compiler_params=pltpu.CompilerParams(needs_layout_passes=False,
                                             use_tc_tiling_on_sc=False,
                                             skip_device_barrier=True),
    )


@functools.cache
def _sc_scatter():
    return pl.kernel(
        _sc_scatter_kernel,
        out_type=jax.ShapeDtypeStruct((NB * N, BW), f32),
        mesh=_mesh(),
        scratch_types=[
            pltpu.VMEM_SHARED((N, BW), f32),
            pltpu.VMEM((SUP, 3, CH), i32),
            pltpu.VMEM((DEPTH, CH), i32),
            pltpu.VMEM((DEPTH, CH, BW), f32),
            pltpu.VMEM((STG, BW), f32),
            pltpu.SemaphoreType.DMA,
            pltpu.SemaphoreType.DMA,
        ],
        compiler_params=pltpu.CompilerParams(needs_layout_passes=False,
                                             use_tc_tiling_on_sc=False,
                                             skip_device_barrier=True),
    )


def _tc_first_body(dega_ref, degb_ref, x_ref, w_ref, dinv_ref,
                   g0_ref, g1_ref, g2_ref, g3_ref):
    deg = dega_ref[...] + degb_ref[...] + 1.0
    dinv = lax.rsqrt(deg)
    dinv_ref[...] = dinv
    hw = jnp.dot(x_ref[...], w_ref[...], preferred_element_type=f32)
    gg = hw * dinv
    g0_ref[...] = gg[:, 0 * BW:1 * BW]
    g1_ref[...] = gg[:, 1 * BW:2 * BW]
    g2_ref[...] = gg[:, 2 * BW:3 * BW]
    g3_ref[...] = gg[:, 3 * BW:4 * BW]


def _tc_mid_body(s0, s1, s2, s3, g0, g1, g2, g3, dinv_ref, b_ref, w_ref,
                 o0, o1, o2, o3):
    dinv = dinv_ref[...]
    hcat = jnp.concatenate(
        [s0[...] + g0[...], s1[...] + g1[...], s2[...] + g2[...], s3[...] + g3[...]],
        axis=1)
    t = jnp.maximum(dinv * hcat + b_ref[...], 0.0)
    hw = jnp.dot(t, w_ref[...], preferred_element_type=f32)
    gg = hw * dinv
    o0[...] = gg[:, 0 * BW:1 * BW]
    o1[...] = gg[:, 1 * BW:2 * BW]
    o2[...] = gg[:, 2 * BW:3 * BW]
    o3[...] = gg[:, 3 * BW:4 * BW]


def _tc_head_body(s0, s1, s2, s3, g0, g1, g2, g3, dinv_ref, b_ref,
                  m1w, m1b, m2w, m2b, m3w, m3b, out_ref):
    dinv = dinv_ref[...]
    hcat = jnp.concatenate(
        [s0[...] + g0[...], s1[...] + g1[...], s2[...] + g2[...], s3[...] + g3[...]],
        axis=1)
    t = jnp.maximum(dinv * hcat + b_ref[...], 0.0)
    h3 = jnp.concatenate([t[:, j * BW:j * BW + 25] for j in range(NB)], axis=1)
    m1 = jnp.maximum(jnp.dot(h3, m1w[...], preferred_element_type=f32) + m1b[...], 0.0)
    m2 = jnp.maximum(jnp.dot(m1, m2w[...], preferred_element_type=f32) + m2b[...], 0.0)
    lg = jnp.dot(m2, m3w[...], preferred_element_type=f32) + m3b[...]
    mx = jnp.max(lg, axis=1, keepdims=True)
    lse = mx + jnp.log(jnp.sum(jnp.exp(lg - mx), axis=1, keepdims=True))
    out_ref[...] = lg - lse


def _row_spec(w):
    return pl.BlockSpec((RB, w), lambda i: (i, 0))


def _full_spec(shape):
    return pl.BlockSpec(shape, lambda i: tuple(0 for _ in shape))


def _tc_first(dega2, degb2, x, w1p):
    return pl.pallas_call(
        _tc_first_body,
        grid=(GRID,),
        in_specs=[_row_spec(1), _row_spec(1), _row_spec(DIN), _full_spec((DIN, NB * BW))],
        out_specs=[_row_spec(1)] + [_row_spec(BW)] * NB,
        out_shape=[jax.ShapeDtypeStruct((N, 1), f32)]
        + [jax.ShapeDtypeStruct((N, BW), f32) for _ in range(NB)],
    )(dega2, degb2, x, w1p)


def _tc_mid(ss, gs, dinv, bp, wp):
    return pl.pallas_call(
        _tc_mid_body,
        grid=(GRID,),
        in_specs=[_row_spec(BW)] * (2 * NB)
        + [_row_spec(1), _full_spec((1, NB * BW)), _full_spec((NB * BW, NB * BW))],
        out_specs=[_row_spec(BW)] * NB,
        out_shape=[jax.ShapeDtypeStruct((N, BW), f32) for _ in range(NB)],
    )(*ss, *gs, dinv, bp, wp)


def _tc_head(ss, gs, dinv, bp, m1w, m1b, m2w, m2b, m3w, m3b):
    return pl.pallas_call(
        _tc_head_body,
        grid=(GRID,),
        in_specs=[_row_spec(BW)] * (2 * NB)
        + [_row_spec(1), _full_spec((1, NB * BW)),
           _full_spec((H, H // 2)), _full_spec((1, H // 2)),
           _full_spec((H // 2, H // 2)), _full_spec((1, H // 2)),
           _full_spec((H // 2, C)), _full_spec((1, C))],
        out_specs=_row_spec(C),
        out_shape=jax.ShapeDtypeStruct((N, C), f32),
    )(*ss, *gs, dinv, bp, m1w, m1b, m2w, m2b, m3w, m3b)


def _pad_w_in(w):
    fi = w.shape[0]
    return jnp.pad(w.reshape(fi, NB, 25), ((0, 0), (0, 0), (0, BW - 25))).reshape(fi, NB * BW)


def _pad_w_both(w):
    w4 = w.reshape(NB, 25, NB, 25)
    w4 = jnp.pad(w4, ((0, 0), (0, BW - 25), (0, 0), (0, BW - 25)))
    return w4.reshape(NB * BW, NB * BW)


def _pad_b(b):
    return jnp.pad(b.reshape(NB, 25), ((0, 0), (0, BW - 25))).reshape(1, NB * BW)


def kernel(x, edge_index, edge_attr, W1, b1, W2, b2, W3, b3,
           M1w, M1b, M2w, M2b, M3w, M3b):
    src = edge_index[0].astype(i32)
    dst = edge_index[1].astype(i32)
    pad = EPAD - E
    srcp = jnp.concatenate([src, jnp.zeros((pad,), i32)])
    dstp = jnp.concatenate([dst, jnp.zeros((pad,), i32)])
    eap = jnp.concatenate([edge_attr.astype(f32), jnp.zeros((pad,), f32)])
    pk = jnp.stack([srcp.reshape(EPAD // CH, CH),
                    dstp.reshape(EPAD // CH, CH),
                    lax.bitcast_convert_type(eap, i32).reshape(EPAD // CH, CH)],
                   axis=1)
    zeros1 = jnp.zeros((N,), f32)
    zeros32 = jnp.zeros((N, BW), f32)

    w1p = _pad_w_in(W1)
    w2p = _pad_w_both(W2)
    w3p = _pad_w_both(W3)
    b1p, b2p, b3p = _pad_b(b1), _pad_b(b2), _pad_b(b3)

    dega, degb = _sc_deg()(dstp, eap, zeros1)
    dinv, *g = _tc_first(dega.reshape(N, 1), degb.reshape(N, 1), x, w1p)

    def edge_scatter(g_list):
        s_all = _sc_scatter()(jnp.concatenate(g_list), pk, zeros32)
        return [s_all[j * N:(j + 1) * N] for j in range(NB)]

    s = edge_scatter(g)
    g = _tc_mid(s, g, dinv, b1p, w2p)
    s = edge_scatter(g)
    g = _tc_mid(s, g, dinv, b2p, w3p)
    s = edge_scatter(g)

    return _tc_head(s, g, dinv, b3p,
                    M1w, M1b.reshape(1, -1), M2w, M2b.reshape(1, -1),
                    M3w, M3b.reshape(1, -1))


# speedup vs baseline: 13.3058x; 1.3661x over previous
import functools

import jax
import jax.numpy as jnp
from jax import lax
from jax.experimental import pallas as pl
from jax.experimental.pallas import tpu as pltpu
from jax.experimental.pallas import tpu_sc as plsc

N = 50000
E = 800000
DIN = 200
H = 100
C = 11

NB = 4
BW = 32
CH = 128
NS = 16
NC = 2
ROWS_PT = N // NS
EPAD = 802816
DEG_CHUNKS = EPAD // (NC * NS * CH)
SC_CHUNKS = EPAD // (NS * CH)

RB = 2000
GRID = N // RB

f32 = jnp.float32
i32 = jnp.int32


@functools.cache
def _mesh():
    return plsc.VectorSubcoreMesh(core_axis_name="c", subcore_axis_name="s")


def _copy_1d_slices(s, src, stage, dst):
    r0 = pl.multiple_of(s * 3128, 8)

    @pl.when(s < NS - 1)
    def _():
        pltpu.sync_copy(src.at[pl.ds(r0, 3128)], stage.at[pl.ds(0, 3128)])
        pltpu.sync_copy(stage.at[pl.ds(0, 3128)], dst.at[pl.ds(r0, 3128)])

    @pl.when(s == NS - 1)
    def _():
        r1 = 3128 * (NS - 1)
        nr = N - r1
        pltpu.sync_copy(src.at[pl.ds(r1, nr)], stage.at[pl.ds(0, nr)])
        pltpu.sync_copy(stage.at[pl.ds(0, nr)], dst.at[pl.ds(r1, nr)])


SUP_D = 14


def _sc_deg_body(pk_hbm, z1_hbm, dega, degb, acc, pkd, ea_ring, stage, sem_d):
    c = lax.axis_index("c")
    s = lax.axis_index("s")
    gtid = c * NS + s
    _copy_1d_slices(s, z1_hbm, stage, acc)
    plsc.subcore_barrier()

    def drain():
        pltpu.make_async_copy(ea_ring.at[0], acc.at[pkd.at[0, 1]],
                              sem_d).wait()

    def superchunk(u, carry1):
        @pl.when(u > 0)
        def _():
            drain()
            drain()

        row0 = gtid * DEG_CHUNKS + u * SUP_D
        pltpu.sync_copy(pk_hbm.at[pl.ds(row0, SUP_D)], pkd)

        def chunk(k, carry2):
            b = lax.rem(k, 4)

            @pl.when(k >= 2)
            def _():
                drain()

            for gg in range(CH // 16):
                sl = pl.ds(gg * 16, 16)
                ea_ring[b, sl] = plsc.bitcast(pkd[k, 2, sl], f32)
            pltpu.make_async_copy(ea_ring.at[b], acc.at[pkd.at[k, 1]],
                                  sem_d).start(add=True)
            return carry2

        lax.fori_loop(0, SUP_D, chunk, 0)
        return carry1

    lax.fori_loop(0, DEG_CHUNKS // SUP_D, superchunk, 0)
    drain()
    drain()
    plsc.subcore_barrier()

    @pl.when(c == 0)
    def _():
        _copy_1d_slices(s, acc, stage, dega)

    @pl.when(c == 1)
    def _():
        _copy_1d_slices(s, acc, stage, degb)


STG = 256


def _staged_rows(src, stage, dst, r0, nrows, dst_base=None):
    d0 = r0 if dst_base is None else dst_base
    for off in range(0, nrows, STG):
        n = min(STG, nrows - off)
        ro = pl.multiple_of(r0 + off, 8)
        do = pl.multiple_of(d0 + off, 8)
        pltpu.sync_copy(src.at[pl.ds(ro, n)], stage.at[pl.ds(0, n)])
        pltpu.sync_copy(stage.at[pl.ds(0, n)], dst.at[pl.ds(do, n)])


def _copy_2d_slices(s, src, stage, dst):
    @pl.when(s < NS - 1)
    def _():
        _staged_rows(src, stage, dst, s * 3128, 3128)

    @pl.when(s == NS - 1)
    def _():
        _staged_rows(src, stage, dst, 3128 * (NS - 1), N - 3128 * (NS - 1))


SUP = 14
SUPERS = SC_CHUNKS // SUP
DEPTH = 4


def _lane_splat(vec16, lane):
    idx = jnp.full((16, 1), lane, i32)
    return lax.gather(
        vec16, idx,
        dimension_numbers=lax.GatherDimensionNumbers(
            offset_dims=(), collapsed_slice_dims=(0,), start_index_map=(0,)),
        slice_sizes=(1,),
        mode=lax.GatherScatterMode.PROMISE_IN_BOUNDS)


def _sc_scatter_kernel(g_flat, pk_hbm, z_hbm, s_all,
                       acc, pk, src_sh, rows_v, stage, sem, sem_s):
    c = lax.axis_index("c")
    s = lax.axis_index("s")

    def drain_scatter():
        pltpu.make_async_copy(rows_v.at[0], acc.at[pk.at[0, 1]], sem_s).wait()

    def col_block(jj, carry0):
        j = c * 2 + jj
        jn = j * N
        _copy_2d_slices(s, z_hbm, stage, acc)
        plsc.subcore_barrier()

        def shift_issue(kt):
            b2 = lax.rem(kt, DEPTH)
            for gg in range(CH // 16):
                sl = pl.ds(gg * 16, 16)
                src_sh[b2, sl] = pk[kt, 0, sl] + jn
            pltpu.async_copy(g_flat.at[src_sh.at[b2]],
                             rows_v.at[lax.rem(kt, DEPTH)], sem)

        def superchunk(u, carry1):
            @pl.when(u > 0)
            def _():
                drain_scatter()
                drain_scatter()

            row0 = s * SC_CHUNKS + u * SUP
            pltpu.sync_copy(pk_hbm.at[pl.ds(row0, SUP)], pk)
            shift_issue(0)
            shift_issue(1)

            def chunk(k, carry2):
                b = lax.rem(k, DEPTH)

                @pl.when(k >= 2)
                def _():
                    drain_scatter()

                @pl.when(k < SUP - 2)
                def _():
                    shift_issue(k + 2)

                pltpu.make_async_copy(g_flat.at[src_sh.at[0]],
                                      rows_v.at[0], sem).wait()
                for e in range(CH):
                    if e % 16 == 0:
                        ea16 = plsc.bitcast(pk[k, 2, pl.ds(e, 16)], f32)
                    sp = _lane_splat(ea16, e % 16)
                    lo = rows_v[b, e, pl.ds(0, 16)]
                    hi = rows_v[b, e, pl.ds(16, 16)]
                    rows_v[b, e, pl.ds(0, 16)] = lo * sp
                    rows_v[b, e, pl.ds(16, 16)] = hi * sp
                pltpu.make_async_copy(rows_v.at[b], acc.at[pk.at[k, 1]],
                                      sem_s).start(add=True)
                return carry2

            lax.fori_loop(0, SUP, chunk, 0)
            return carry1

        lax.fori_loop(0, SUPERS, superchunk, 0)
        drain_scatter()
        drain_scatter()
        plsc.subcore_barrier()

        @pl.when(s < NS - 1)
        def _():
            _staged_rows(acc, stage, s_all, s * 3128, 3128,
                         dst_base=j * N + s * 3128)

        @pl.when(s == NS - 1)
        def _():
            _staged_rows(acc, stage, s_all, 3128 * (NS - 1),
                         N - 3128 * (NS - 1),
                         dst_base=j * N + 3128 * (NS - 1))

        plsc.subcore_barrier()
        return carry0

    lax.fori_loop(0, 2, col_block, 0)


@functools.cache
def _sc_deg():
    return pl.kernel(
        _sc_deg_body,
        out_type=[jax.ShapeDtypeStruct((N,), f32),
                  jax.ShapeDtypeStruct((N,), f32)],
        mesh=_mesh(),
        scratch_types=[
            pltpu.VMEM_SHARED((N,), f32),
            pltpu.VMEM((SUP_D, 3, CH), i32),
            pltpu.VMEM((4, CH), f32),
            pltpu.VMEM((3128,), f32),
            pltpu.SemaphoreType.DMA,
        ],
        compiler_params=pltpu.CompilerParams(needs_layout_passes=False,
                                             use_tc_tiling_on_sc=False),
    )


@functools.cache
def _sc_scatter():
    return pl.kernel(
        _sc_scatter_kernel,
        out_type=jax.ShapeDtypeStruct((NB * N, BW), f32),
        mesh=_mesh(),
        scratch_types=[
            pltpu.VMEM_SHARED((N, BW), f32),
            pltpu.VMEM((SUP, 3, CH), i32),
            pltpu.VMEM((DEPTH, CH), i32),
            pltpu.VMEM((DEPTH, CH, BW), f32),
            pltpu.VMEM((STG, BW), f32),
            pltpu.SemaphoreType.DMA,
            pltpu.SemaphoreType.DMA,
        ],
        compiler_params=pltpu.CompilerParams(needs_layout_passes=False,
                                             use_tc_tiling_on_sc=False),
    )


def _tc_first_body(dega_ref, degb_ref, x_ref, w_ref, dinv_ref, g4_ref):
    deg = dega_ref[...] + degb_ref[...] + 1.0
    dinv = lax.rsqrt(deg)
    dinv_ref[...] = dinv
    hw = jnp.dot(x_ref[...], w_ref[...], preferred_element_type=f32)
    gg = hw * dinv
    for j in range(NB):
        g4_ref[j] = gg[:, j * BW:(j + 1) * BW]


def _tc_mid_body(s4, g4, dinv_ref, b_ref, w_ref, o4):
    dinv = dinv_ref[...]
    hcat = jnp.concatenate([s4[j] + g4[j] for j in range(NB)], axis=1)
    t = jnp.maximum(dinv * hcat + b_ref[...], 0.0)
    hw = jnp.dot(t, w_ref[...], preferred_element_type=f32)
    gg = hw * dinv
    for j in range(NB):
        o4[j] = gg[:, j * BW:(j + 1) * BW]


def _tc_head_body(s4, g4, dinv_ref, b_ref,
                  m1w, m1b, m2w, m2b, m3w, m3b, out_ref):
    dinv = dinv_ref[...]
    hcat = jnp.concatenate([s4[j] + g4[j] for j in range(NB)], axis=1)
    t = jnp.maximum(dinv * hcat + b_ref[...], 0.0)
    h3 = jnp.concatenate([t[:, j * BW:j * BW + 25] for j in range(NB)], axis=1)
    m1 = jnp.maximum(jnp.dot(h3, m1w[...], preferred_element_type=f32) + m1b[...], 0.0)
    m2 = jnp.maximum(jnp.dot(m1, m2w[...], preferred_element_type=f32) + m2b[...], 0.0)
    lg = jnp.dot(m2, m3w[...], preferred_element_type=f32) + m3b[...]
    mx = jnp.max(lg, axis=1, keepdims=True)
    lse = mx + jnp.log(jnp.sum(jnp.exp(lg - mx), axis=1, keepdims=True))
    out_ref[...] = lg - lse


def _row_spec(w):
    return pl.BlockSpec((RB, w), lambda i: (i, 0))


def _full_spec(shape):
    return pl.BlockSpec(shape, lambda i: tuple(0 for _ in shape))


def _g4_spec():
    return pl.BlockSpec((NB, RB, BW), lambda i: (0, i, 0))


def _tc_first(dega2, degb2, x, w1p):
    return pl.pallas_call(
        _tc_first_body,
        grid=(GRID,),
        in_specs=[_row_spec(1), _row_spec(1), _row_spec(DIN), _full_spec((DIN, NB * BW))],
        out_specs=[_row_spec(1), _g4_spec()],
        out_shape=[jax.ShapeDtypeStruct((N, 1), f32),
                   jax.ShapeDtypeStruct((NB, N, BW), f32)],
    )(dega2, degb2, x, w1p)


def _tc_mid(s4, g4, dinv, bp, wp):
    return pl.pallas_call(
        _tc_mid_body,
        grid=(GRID,),
        in_specs=[_g4_spec(), _g4_spec(), _row_spec(1),
                  _full_spec((1, NB * BW)), _full_spec((NB * BW, NB * BW))],
        out_specs=_g4_spec(),
        out_shape=jax.ShapeDtypeStruct((NB, N, BW), f32),
    )(s4, g4, dinv, bp, wp)


def _tc_head(s4, g4, dinv, bp, m1w, m1b, m2w, m2b, m3w, m3b):
    return pl.pallas_call(
        _tc_head_body,
        grid=(GRID,),
        in_specs=[_g4_spec(), _g4_spec(), _row_spec(1),
                  _full_spec((1, NB * BW)),
                  _full_spec((H, H // 2)), _full_spec((1, H // 2)),
                  _full_spec((H // 2, H // 2)), _full_spec((1, H // 2)),
                  _full_spec((H // 2, C)), _full_spec((1, C))],
        out_specs=_row_spec(C),
        out_shape=jax.ShapeDtypeStruct((N, C), f32),
    )(s4, g4, dinv, bp, m1w, m1b, m2w, m2b, m3w, m3b)


def _pad_w_in(w):
    fi = w.shape[0]
    return jnp.pad(w.reshape(fi, NB, 25), ((0, 0), (0, 0), (0, BW - 25))).reshape(fi, NB * BW)


def _pad_w_both(w):
    w4 = w.reshape(NB, 25, NB, 25)
    w4 = jnp.pad(w4, ((0, 0), (0, BW - 25), (0, 0), (0, BW - 25)))
    return w4.reshape(NB * BW, NB * BW)


def _pad_b(b):
    return jnp.pad(b.reshape(NB, 25), ((0, 0), (0, BW - 25))).reshape(1, NB * BW)


def kernel(x, edge_index, edge_attr, W1, b1, W2, b2, W3, b3,
           M1w, M1b, M2w, M2b, M3w, M3b):
    src = edge_index[0].astype(i32)
    dst = edge_index[1].astype(i32)
    pad = EPAD - E
    srcp = jnp.concatenate([src, jnp.zeros((pad,), i32)])
    dstp = jnp.concatenate([dst, jnp.zeros((pad,), i32)])
    eap = jnp.concatenate([edge_attr.astype(f32), jnp.zeros((pad,), f32)])
    pk = jnp.stack([srcp.reshape(EPAD // CH, CH),
                    dstp.reshape(EPAD // CH, CH),
                    lax.bitcast_convert_type(eap, i32).reshape(EPAD // CH, CH)],
                   axis=1)
    zeros1 = jnp.zeros((N,), f32)
    zeros32 = jnp.zeros((N, BW), f32)

    w1p = _pad_w_in(W1)
    w2p = _pad_w_both(W2)
    w3p = _pad_w_both(W3)
    b1p, b2p, b3p = _pad_b(b1), _pad_b(b2), _pad_b(b3)

    dega, degb = _sc_deg()(pk, zeros1)
    dinv, g = _tc_first(dega.reshape(N, 1), degb.reshape(N, 1), x, w1p)

    def edge_scatter(g4):
        s_all = _sc_scatter()(g4.reshape(NB * N, BW), pk, zeros32)
        return s_all.reshape(NB, N, BW)

    s = edge_scatter(g)
    g = _tc_mid(s, g, dinv, b1p, w2p)
    s = edge_scatter(g)
    g = _tc_mid(s, g, dinv, b2p, w3p)
    s = edge_scatter(g)

    return _tc_head(s, g, dinv, b3p,
                    M1w, M1b.reshape(1, -1), M2w, M2b.reshape(1, -1),
                    M3w, M3b.reshape(1, -1))
